# Initial kernel scaffold; baseline (speedup 1.0000x reference)
import jax
import jax.numpy as jnp
from jax.experimental import pallas as pl


def kernel(hidden_states, Wq, Wk, Wv, Wo, Wqi, Wqo, Wvi, Wvo, index_keys, index_values):
    raise NotImplementedError("write your pallas kernel here")


# speedup vs baseline: 2.8521x; 2.8521x over previous
import functools
import math

import jax
import jax.numpy as jnp
from jax import lax
from jax.experimental import pallas as pl
from jax.experimental.pallas import tpu as pltpu
from jax.experimental.pallas import tpu_sc as plsc

_B, _S, _H = 1, 2048, 2048
_NH = 16
_HD = _H // _NH
_R = 16
_D = 64
_K = 100000
_TOPK = 5
_SCALING = _R

_NEG = -1e30
_IBIG = 2**31 - 1

_QROWS = 416
_KBLK = 2000


def _qkv_kernel(x_ref, w_ref, o_ref):
    o_ref[...] = lax.dot_general(
        x_ref[...], w_ref[...], (((1,), (1,)), ((), ())),
        preferred_element_type=jnp.float32)


def _qkv_proj(x, wcat):
    cb = 512
    return pl.pallas_call(
        _qkv_kernel,
        grid=(3 * _H // cb,),
        in_specs=[
            pl.BlockSpec((_S, _H), lambda i: (0, 0)),
            pl.BlockSpec((cb, _H), lambda i: (i, 0)),
        ],
        out_specs=pl.BlockSpec((_S, cb), lambda i: (0, i)),
        out_shape=jax.ShapeDtypeStruct((_S, 3 * _H), jnp.float32),
    )(x, wcat)


def _qa_kernel(x_ref, wqi_ref, wqo_ref, o_ref):
    t = lax.dot_general(
        x_ref[...], wqi_ref[...], (((1,), (1,)), ((), ())),
        preferred_element_type=jnp.float32)
    o_ref[...] = lax.dot_general(
        t, wqo_ref[...], (((1,), (1,)), ((), ())),
        preferred_element_type=jnp.float32)


def _qa_proj(x, wqi, wqo):
    bs = 256
    return pl.pallas_call(
        _qa_kernel,
        grid=(_S // bs,),
        in_specs=[
            pl.BlockSpec((bs, _H), lambda i: (i, 0)),
            pl.BlockSpec((_R, _H), lambda i: (0, 0)),
            pl.BlockSpec((_D, _R), lambda i: (0, 0)),
        ],
        out_specs=pl.BlockSpec((bs, _D), lambda i: (i, 0)),
        out_shape=jax.ShapeDtypeStruct((_S, _D), jnp.float32),
    )(x, wqi, wqo)


def _topk_kernel(q_ref, keys_ref, oidx_ref, rv_ref, ri_ref):
    i = pl.program_id(0)
    nb = pl.num_programs(0)

    @pl.when(i == 0)
    def _():
        rv_ref[...] = jnp.full((_QROWS, 8), _NEG, jnp.float32)
        ri_ref[...] = jnp.full((_QROWS, 8), _IBIG, jnp.int32)

    s = lax.dot_general(
        q_ref[...], keys_ref[...], (((1,), (1,)), ((), ())),
        preferred_element_type=jnp.float32)
    col = lax.broadcasted_iota(jnp.int32, (_QROWS, _KBLK), 1) + i * _KBLK
    lane8 = lax.broadcasted_iota(jnp.int32, (_QROWS, 8), 1)

    bv = jnp.full((_QROWS, 8), _NEG, jnp.float32)
    bi = jnp.full((_QROWS, 8), _IBIG, jnp.int32)
    for t in range(_TOPK):
        v = jnp.max(s, axis=1, keepdims=True)
        idx = jnp.min(jnp.where(s == v, col, _IBIG), axis=1, keepdims=True)
        bv = jnp.where(lane8 == t, v, bv)
        bi = jnp.where(lane8 == t, idx, bi)
        s = jnp.where(col == idx, _NEG, s)

    cv = jnp.concatenate([rv_ref[...], bv], axis=1)
    ci = jnp.concatenate([ri_ref[...], bi], axis=1)
    nv = jnp.full((_QROWS, 8), _NEG, jnp.float32)
    ni = jnp.full((_QROWS, 8), _IBIG, jnp.int32)
    for t in range(_TOPK):
        v = jnp.max(cv, axis=1, keepdims=True)
        idx = jnp.min(jnp.where(cv == v, ci, _IBIG), axis=1, keepdims=True)
        nv = jnp.where(lane8 == t, v, nv)
        ni = jnp.where(lane8 == t, idx, ni)
        cv = jnp.where(ci == idx, _NEG, cv)
    rv_ref[...] = nv
    ri_ref[...] = ni

    @pl.when(i == nb - 1)
    def _():
        oidx_ref[...] = ri_ref[...]


def _retrieval_topk(qa_head, index_keys):
    return pl.pallas_call(
        _topk_kernel,
        grid=(_K // _KBLK,),
        in_specs=[
            pl.BlockSpec((_QROWS, _D), lambda i: (0, 0)),
            pl.BlockSpec((_KBLK, _D), lambda i: (i, 0)),
        ],
        out_specs=pl.BlockSpec((_QROWS, 8), lambda i: (0, 0)),
        out_shape=jax.ShapeDtypeStruct((_QROWS, 8), jnp.int32),
        scratch_shapes=[
            pltpu.VMEM((_QROWS, 8), jnp.float32),
            pltpu.VMEM((_QROWS, 8), jnp.int32),
        ],
    )(qa_head, index_keys)


def _sc_gather(catkv, idx):
    n = idx.shape[0]
    nw = 32
    per = n // nw
    mesh = plsc.VectorSubcoreMesh(core_axis_name="c", subcore_axis_name="s")

    @functools.partial(
        pl.kernel,
        mesh=mesh,
        out_type=jax.ShapeDtypeStruct((n, 2 * _D), jnp.float32),
        scratch_types=[
            pltpu.VMEM((per,), jnp.int32),
            pltpu.VMEM((per, 2 * _D), jnp.float32),
            pltpu.SemaphoreType.DMA,
        ],
    )
    def gk(cat_hbm, idx_hbm, o_hbm, idx_v, rows_v, sem):
        wid = lax.axis_index("s") * 2 + lax.axis_index("c")
        base = wid * per
        pltpu.sync_copy(idx_hbm.at[pl.ds(base, per)], idx_v)
        pltpu.async_copy(cat_hbm.at[idx_v], rows_v, sem).wait()
        pltpu.sync_copy(rows_v, o_hbm.at[pl.ds(base, per)])

    return gk(catkv, idx)


def _flash_kernel(q_ref, k_ref, v_ref, o_ref, acc_ref, m_ref, l_ref,
                  *, scale, bq, bk, hd, split_kv=False):
    qi = pl.program_id(1)
    ki = pl.program_id(2)

    @pl.when(ki == 0)
    def _():
        m_ref[...] = jnp.full((bq, 128), _NEG, jnp.float32)
        l_ref[...] = jnp.zeros((bq, 128), jnp.float32)
        acc_ref[...] = jnp.zeros((bq, hd), jnp.float32)

    @pl.when(ki <= qi)
    def _():
        if split_kv:
            kv = k_ref[...]
            kb = kv[:, :hd]
            vb = kv[:, hd:]
        else:
            kb = k_ref[...]
            vb = v_ref[...]
        s = lax.dot_general(
            q_ref[...], kb, (((1,), (1,)), ((), ())),
            preferred_element_type=jnp.float32) * scale
        row = lax.broadcasted_iota(jnp.int32, (bq, bk), 0) + qi * bq
        colg = lax.broadcasted_iota(jnp.int32, (bq, bk), 1) + ki * bk
        s = jnp.where(colg <= row, s, _NEG)

        m_prev = m_ref[...][:, :1]
        l_prev = l_ref[...][:, :1]
        m_cur = jnp.max(s, axis=1, keepdims=True)
        m_new = jnp.maximum(m_prev, m_cur)
        alpha = jnp.exp(m_prev - m_new)
        p = jnp.exp(s - m_new)
        l_new = l_prev * alpha + jnp.sum(p, axis=1, keepdims=True)
        acc_ref[...] = acc_ref[...] * alpha + lax.dot_general(
            p, vb, (((1,), (0,)), ((), ())),
            preferred_element_type=jnp.float32)
        m_ref[...] = jnp.broadcast_to(m_new, (bq, 128))
        l_ref[...] = jnp.broadcast_to(l_new, (bq, 128))

    @pl.when(ki == qi)
    def _():
        o_ref[...] = acc_ref[...] / l_ref[...][:, :1]


def _base_attention(qkv):
    bq = bk = 256
    nq = _S // bq
    kern = functools.partial(
        _flash_kernel, scale=1.0 / math.sqrt(_HD), bq=bq, bk=bk, hd=_HD)
    return pl.pallas_call(
        kern,
        grid=(_NH, nq, nq),
        in_specs=[
            pl.BlockSpec((bq, _HD), lambda h, qi, ki: (qi, h)),
            pl.BlockSpec((bk, _HD),
                         lambda h, qi, ki: (jnp.minimum(ki, qi), _NH + h)),
            pl.BlockSpec((bk, _HD),
                         lambda h, qi, ki: (jnp.minimum(ki, qi), 2 * _NH + h)),
        ],
        out_specs=pl.BlockSpec((bq, _HD), lambda h, qi, ki: (qi, h)),
        out_shape=jax.ShapeDtypeStruct((_S, _H), jnp.float32),
        scratch_shapes=[
            pltpu.VMEM((bq, _HD), jnp.float32),
            pltpu.VMEM((bq, 128), jnp.float32),
            pltpu.VMEM((bq, 128), jnp.float32),
        ],
    )(qkv, qkv, qkv)


def _adapter_attention(qa, kvg):
    bq = bk = 256
    nq = _S // bq
    kern = functools.partial(
        _flash_kernel, scale=1.0 / math.sqrt(_D), bq=bq, bk=bk, hd=_D,
        split_kv=True)
    return pl.pallas_call(
        kern,
        grid=(1, nq, nq),
        in_specs=[
            pl.BlockSpec((bq, _D), lambda h, qi, ki: (qi, 0)),
            pl.BlockSpec((bk, 2 * _D),
                         lambda h, qi, ki: (jnp.minimum(ki, qi), 0)),
            pl.BlockSpec((bk, 2 * _D),
                         lambda h, qi, ki: (jnp.minimum(ki, qi), 0)),
        ],
        out_specs=pl.BlockSpec((bq, _D), lambda h, qi, ki: (qi, 0)),
        out_shape=jax.ShapeDtypeStruct((_S, _D), jnp.float32),
        scratch_shapes=[
            pltpu.VMEM((bq, _D), jnp.float32),
            pltpu.VMEM((bq, 128), jnp.float32),
            pltpu.VMEM((bq, 128), jnp.float32),
        ],
    )(qa, kvg, kvg)


def _final_kernel(attn_ref, wo_ref, mome_ref, wvi_ref, wvo_ref, o_ref):
    base = lax.dot_general(
        attn_ref[...], wo_ref[...], (((1,), (1,)), ((), ())),
        preferred_element_type=jnp.float32)
    t = lax.dot_general(
        mome_ref[...], wvi_ref[...], (((1,), (1,)), ((), ())),
        preferred_element_type=jnp.float32)
    ad = lax.dot_general(
        t, wvo_ref[...], (((1,), (1,)), ((), ())),
        preferred_element_type=jnp.float32)
    o_ref[...] = base + ad * float(_SCALING)


def _final_proj(attn, wo, mome, wvi, wvo):
    cb = 512
    return pl.pallas_call(
        _final_kernel,
        grid=(_H // cb,),
        in_specs=[
            pl.BlockSpec((_S, _H), lambda i: (0, 0)),
            pl.BlockSpec((cb, _H), lambda i: (i, 0)),
            pl.BlockSpec((_S, _D), lambda i: (0, 0)),
            pl.BlockSpec((_R, _D), lambda i: (0, 0)),
            pl.BlockSpec((cb, _R), lambda i: (i, 0)),
        ],
        out_specs=pl.BlockSpec((_S, cb), lambda i: (0, i)),
        out_shape=jax.ShapeDtypeStruct((_S, _H), jnp.float32),
    )(attn, wo, mome, wvi, wvo)


def kernel(hidden_states, Wq, Wk, Wv, Wo, Wqi, Wqo, Wvi, Wvo,
           index_keys, index_values):
    x = hidden_states.reshape(_S, _H)
    wcat = jnp.concatenate([Wq, Wk, Wv], axis=0)

    qkv = _qkv_proj(x, wcat)
    qa = _qa_proj(x, Wqi, Wqo)

    topidx = _retrieval_topk(qa[:_QROWS], index_keys)
    flat_idx = topidx[:410, :_TOPK].reshape(-1)[:_S]

    catkv = jnp.concatenate([index_keys, index_values], axis=1)
    kvg = _sc_gather(catkv, flat_idx)

    attn = _base_attention(qkv)
    mome = _adapter_attention(qa, kvg)

    out = _final_proj(attn, Wo, mome, Wvi, Wvo)
    return out.reshape(_B, _S, _H)


# speedup vs baseline: 3.4071x; 1.1946x over previous
import functools
import math

import jax
import jax.numpy as jnp
from jax import lax
from jax.experimental import pallas as pl
from jax.experimental.pallas import tpu as pltpu
from jax.experimental.pallas import tpu_sc as plsc

_B, _S, _H = 1, 2048, 2048
_NH = 16
_HD = _H // _NH
_R = 16
_D = 64
_K = 100000
_TOPK = 5
_SCALING = _R

_NEG = -1e30
_IBIG = 2**31 - 1

_QROWS = 416
_KBLK = 2000


def _qkv_kernel(x_ref, w_ref, o_ref):
    o_ref[...] = lax.dot_general(
        x_ref[...], w_ref[...], (((1,), (1,)), ((), ())),
        preferred_element_type=jnp.float32)


def _qkv_proj(x, wcat):
    cb = 512
    return pl.pallas_call(
        _qkv_kernel,
        grid=(3 * _H // cb,),
        in_specs=[
            pl.BlockSpec((_S, _H), lambda i: (0, 0)),
            pl.BlockSpec((cb, _H), lambda i: (i, 0)),
        ],
        out_specs=pl.BlockSpec((_S, cb), lambda i: (0, i)),
        out_shape=jax.ShapeDtypeStruct((_S, 3 * _H), jnp.float32),
    )(x, wcat)


def _qa_kernel(x_ref, wqi_ref, wqo_ref, o_ref):
    t = lax.dot_general(
        x_ref[...], wqi_ref[...], (((1,), (1,)), ((), ())),
        preferred_element_type=jnp.float32)
    o_ref[...] = lax.dot_general(
        t, wqo_ref[...], (((1,), (1,)), ((), ())),
        preferred_element_type=jnp.float32)


def _qa_proj(x, wqi, wqo):
    bs = 256
    return pl.pallas_call(
        _qa_kernel,
        grid=(_S // bs,),
        in_specs=[
            pl.BlockSpec((bs, _H), lambda i: (i, 0)),
            pl.BlockSpec((_R, _H), lambda i: (0, 0)),
            pl.BlockSpec((_D, _R), lambda i: (0, 0)),
        ],
        out_specs=pl.BlockSpec((bs, _D), lambda i: (i, 0)),
        out_shape=jax.ShapeDtypeStruct((_S, _D), jnp.float32),
    )(x, wqi, wqo)


def _topk_kernel(q_ref, keys_ref, oidx_ref, rv_ref, ri_ref):
    i = pl.program_id(0)
    nb = pl.num_programs(0)

    @pl.when(i == 0)
    def _():
        rv_ref[...] = jnp.full((_QROWS, 8), _NEG, jnp.float32)
        ri_ref[...] = jnp.full((_QROWS, 8), _IBIG, jnp.int32)

    s = lax.dot_general(
        q_ref[...], keys_ref[...], (((1,), (1,)), ((), ())),
        preferred_element_type=jnp.float32)
    col = lax.broadcasted_iota(jnp.int32, (_QROWS, _KBLK), 1) + i * _KBLK
    lane8 = lax.broadcasted_iota(jnp.int32, (_QROWS, 8), 1)

    bv = jnp.full((_QROWS, 8), _NEG, jnp.float32)
    bi = jnp.full((_QROWS, 8), _IBIG, jnp.int32)
    for t in range(_TOPK):
        v = jnp.max(s, axis=1, keepdims=True)
        idx = jnp.min(jnp.where(s == v, col, _IBIG), axis=1, keepdims=True)
        bv = jnp.where(lane8 == t, v, bv)
        bi = jnp.where(lane8 == t, idx, bi)
        s = jnp.where(col == idx, _NEG, s)

    cv = jnp.concatenate([rv_ref[...], bv], axis=1)
    ci = jnp.concatenate([ri_ref[...], bi], axis=1)
    nv = jnp.full((_QROWS, 8), _NEG, jnp.float32)
    ni = jnp.full((_QROWS, 8), _IBIG, jnp.int32)
    for t in range(_TOPK):
        v = jnp.max(cv, axis=1, keepdims=True)
        idx = jnp.min(jnp.where(cv == v, ci, _IBIG), axis=1, keepdims=True)
        nv = jnp.where(lane8 == t, v, nv)
        ni = jnp.where(lane8 == t, idx, ni)
        cv = jnp.where(ci == idx, _NEG, cv)
    rv_ref[...] = nv
    ri_ref[...] = ni

    @pl.when(i == nb - 1)
    def _():
        oidx_ref[...] = ri_ref[...]


def _retrieval_topk(qa_head, index_keys):
    return pl.pallas_call(
        _topk_kernel,
        grid=(_K // _KBLK,),
        in_specs=[
            pl.BlockSpec((_QROWS, _D), lambda i: (0, 0)),
            pl.BlockSpec((_KBLK, _D), lambda i: (i, 0)),
        ],
        out_specs=pl.BlockSpec((_QROWS, 8), lambda i: (0, 0)),
        out_shape=jax.ShapeDtypeStruct((_QROWS, 8), jnp.int32),
        scratch_shapes=[
            pltpu.VMEM((_QROWS, 8), jnp.float32),
            pltpu.VMEM((_QROWS, 8), jnp.int32),
        ],
    )(qa_head, index_keys)


def _sc_gather(catkv, idx):
    n = idx.shape[0]
    nw = 32
    per = n // nw
    mesh = plsc.VectorSubcoreMesh(core_axis_name="c", subcore_axis_name="s")

    @functools.partial(
        pl.kernel,
        mesh=mesh,
        out_type=jax.ShapeDtypeStruct((n, 2 * _D), jnp.float32),
        scratch_types=[
            pltpu.VMEM((per,), jnp.int32),
            pltpu.VMEM((per, 2 * _D), jnp.float32),
            pltpu.SemaphoreType.DMA,
        ],
    )
    def gk(cat_hbm, idx_hbm, o_hbm, idx_v, rows_v, sem):
        wid = lax.axis_index("s") * 2 + lax.axis_index("c")
        base = wid * per
        pltpu.sync_copy(idx_hbm.at[pl.ds(base, per)], idx_v)
        pltpu.async_copy(cat_hbm.at[idx_v], rows_v, sem).wait()
        pltpu.sync_copy(rows_v, o_hbm.at[pl.ds(base, per)])

    return gk(catkv, idx)


def _flash_kernel(q_ref, k_ref, v_ref, o_ref, acc_ref, m_ref, l_ref,
                  *, scale, bq, bk, hd, split_kv=False):
    qi = pl.program_id(1)
    ki = pl.program_id(2)

    @pl.when(ki == 0)
    def _():
        m_ref[...] = jnp.full((bq, 128), _NEG, jnp.float32)
        l_ref[...] = jnp.zeros((bq, 128), jnp.float32)
        acc_ref[...] = jnp.zeros((bq, hd), jnp.float32)

    def step(masked):
        if split_kv:
            kv = k_ref[...]
            kb = kv[:, :hd]
            vb = kv[:, hd:]
        else:
            kb = k_ref[...]
            vb = v_ref[...]
        s = lax.dot_general(
            q_ref[...].astype(jnp.bfloat16), kb.astype(jnp.bfloat16),
            (((1,), (1,)), ((), ())),
            preferred_element_type=jnp.float32) * scale
        if masked:
            row = lax.broadcasted_iota(jnp.int32, (bq, bk), 0)
            colg = lax.broadcasted_iota(jnp.int32, (bq, bk), 1)
            s = jnp.where(colg <= row, s, _NEG)

        m_prev = m_ref[...][:, :1]
        l_prev = l_ref[...][:, :1]
        m_cur = jnp.max(s, axis=1, keepdims=True)
        m_new = jnp.maximum(m_prev, m_cur)
        alpha = jnp.exp(m_prev - m_new)
        p = jnp.exp(s - m_new)
        l_new = l_prev * alpha + jnp.sum(p, axis=1, keepdims=True)
        acc_ref[...] = acc_ref[...] * alpha + lax.dot_general(
            p.astype(jnp.bfloat16), vb.astype(jnp.bfloat16),
            (((1,), (0,)), ((), ())),
            preferred_element_type=jnp.float32)
        m_ref[...] = jnp.broadcast_to(m_new, (bq, 128))
        l_ref[...] = jnp.broadcast_to(l_new, (bq, 128))
        if masked:
            o_ref[...] = acc_ref[...] / l_ref[...][:, :1]

    @pl.when(ki < qi)
    def _():
        step(False)

    @pl.when(ki == qi)
    def _():
        step(True)


def _base_attention(qkv):
    bq = bk = 512
    nq = _S // bq
    kern = functools.partial(
        _flash_kernel, scale=1.0 / math.sqrt(_HD), bq=bq, bk=bk, hd=_HD)
    return pl.pallas_call(
        kern,
        grid=(_NH, nq, nq),
        in_specs=[
            pl.BlockSpec((bq, _HD), lambda h, qi, ki: (qi, h)),
            pl.BlockSpec((bk, _HD),
                         lambda h, qi, ki: (jnp.minimum(ki, qi), _NH + h)),
            pl.BlockSpec((bk, _HD),
                         lambda h, qi, ki: (jnp.minimum(ki, qi), 2 * _NH + h)),
        ],
        out_specs=pl.BlockSpec((bq, _HD), lambda h, qi, ki: (qi, h)),
        out_shape=jax.ShapeDtypeStruct((_S, _H), jnp.float32),
        scratch_shapes=[
            pltpu.VMEM((bq, _HD), jnp.float32),
            pltpu.VMEM((bq, 128), jnp.float32),
            pltpu.VMEM((bq, 128), jnp.float32),
        ],
    )(qkv, qkv, qkv)


def _adapter_attention(qa, kvg):
    bq = bk = 512
    nq = _S // bq
    kern = functools.partial(
        _flash_kernel, scale=1.0 / math.sqrt(_D), bq=bq, bk=bk, hd=_D,
        split_kv=True)
    return pl.pallas_call(
        kern,
        grid=(1, nq, nq),
        in_specs=[
            pl.BlockSpec((bq, _D), lambda h, qi, ki: (qi, 0)),
            pl.BlockSpec((bk, 2 * _D),
                         lambda h, qi, ki: (jnp.minimum(ki, qi), 0)),
            pl.BlockSpec((bk, 2 * _D),
                         lambda h, qi, ki: (jnp.minimum(ki, qi), 0)),
        ],
        out_specs=pl.BlockSpec((bq, _D), lambda h, qi, ki: (qi, 0)),
        out_shape=jax.ShapeDtypeStruct((_S, _D), jnp.float32),
        scratch_shapes=[
            pltpu.VMEM((bq, _D), jnp.float32),
            pltpu.VMEM((bq, 128), jnp.float32),
            pltpu.VMEM((bq, 128), jnp.float32),
        ],
    )(qa, kvg, kvg)


def _final_kernel(attn_ref, wo_ref, mome_ref, wvi_ref, wvo_ref, o_ref):
    base = lax.dot_general(
        attn_ref[...], wo_ref[...], (((1,), (1,)), ((), ())),
        preferred_element_type=jnp.float32)
    t = lax.dot_general(
        mome_ref[...], wvi_ref[...], (((1,), (1,)), ((), ())),
        preferred_element_type=jnp.float32)
    ad = lax.dot_general(
        t, wvo_ref[...], (((1,), (1,)), ((), ())),
        preferred_element_type=jnp.float32)
    o_ref[...] = base + ad * float(_SCALING)


def _final_proj(attn, wo, mome, wvi, wvo):
    cb = 512
    return pl.pallas_call(
        _final_kernel,
        grid=(_H // cb,),
        in_specs=[
            pl.BlockSpec((_S, _H), lambda i: (0, 0)),
            pl.BlockSpec((cb, _H), lambda i: (i, 0)),
            pl.BlockSpec((_S, _D), lambda i: (0, 0)),
            pl.BlockSpec((_R, _D), lambda i: (0, 0)),
            pl.BlockSpec((cb, _R), lambda i: (i, 0)),
        ],
        out_specs=pl.BlockSpec((_S, cb), lambda i: (0, i)),
        out_shape=jax.ShapeDtypeStruct((_S, _H), jnp.float32),
    )(attn, wo, mome, wvi, wvo)


def kernel(hidden_states, Wq, Wk, Wv, Wo, Wqi, Wqo, Wvi, Wvo,
           index_keys, index_values):
    x = hidden_states.reshape(_S, _H)
    wcat = jnp.concatenate([Wq, Wk, Wv], axis=0)

    qkv = _qkv_proj(x.astype(jnp.bfloat16), wcat.astype(jnp.bfloat16))
    qa = _qa_proj(x, Wqi, Wqo)

    topidx = _retrieval_topk(qa[:_QROWS], index_keys)
    flat_idx = topidx[:410, :_TOPK].reshape(-1)[:_S]

    catkv = jnp.concatenate([index_keys, index_values], axis=1)
    kvg = _sc_gather(catkv, flat_idx)

    attn = _base_attention(qkv)
    mome = _adapter_attention(qa, kvg)

    out = _final_proj(attn.astype(jnp.bfloat16), Wo.astype(jnp.bfloat16),
                      mome, Wvi, Wvo)
    return out.reshape(_B, _S, _H)


# speedup vs baseline: 3.6753x; 1.0787x over previous
import functools
import math

import jax
import jax.numpy as jnp
from jax import lax
from jax.experimental import pallas as pl
from jax.experimental.pallas import tpu as pltpu
from jax.experimental.pallas import tpu_sc as plsc

_B, _S, _H = 1, 2048, 2048
_NH = 16
_HD = _H // _NH
_R = 16
_D = 64
_K = 100000
_TOPK = 5
_SCALING = _R

_NEG = -1e30
_IBIG = 2**31 - 1

_QROWS = 416
_KBLK = 2000


def _qkv_kernel(x_ref, wq_ref, wk_ref, wv_ref, q_ref, k_ref, v_ref):
    for w_ref, o_ref in ((wq_ref, q_ref), (wk_ref, k_ref), (wv_ref, v_ref)):
        o_ref[...] = lax.dot_general(
            x_ref[...], w_ref[...], (((1,), (1,)), ((), ())),
            preferred_element_type=jnp.float32)


def _qkv_proj(x, wq, wk, wv):
    cb = 512
    wspec = pl.BlockSpec((cb, _H), lambda i: (i, 0))
    ospec = pl.BlockSpec((_S, cb), lambda i: (0, i))
    oshape = jax.ShapeDtypeStruct((_S, _H), jnp.float32)
    return pl.pallas_call(
        _qkv_kernel,
        grid=(_H // cb,),
        in_specs=[pl.BlockSpec((_S, _H), lambda i: (0, 0)),
                  wspec, wspec, wspec],
        out_specs=[ospec, ospec, ospec],
        out_shape=[oshape, oshape, oshape],
    )(x, wq, wk, wv)


def _qa_kernel(x_ref, wqi_ref, wqo_ref, o_ref):
    t = lax.dot_general(
        x_ref[...], wqi_ref[...], (((1,), (1,)), ((), ())),
        preferred_element_type=jnp.float32)
    o_ref[...] = lax.dot_general(
        t, wqo_ref[...], (((1,), (1,)), ((), ())),
        preferred_element_type=jnp.float32)


def _qa_proj(x, wqi, wqo):
    bs = 256
    return pl.pallas_call(
        _qa_kernel,
        grid=(_S // bs,),
        in_specs=[
            pl.BlockSpec((bs, _H), lambda i: (i, 0)),
            pl.BlockSpec((_R, _H), lambda i: (0, 0)),
            pl.BlockSpec((_D, _R), lambda i: (0, 0)),
        ],
        out_specs=pl.BlockSpec((bs, _D), lambda i: (i, 0)),
        out_shape=jax.ShapeDtypeStruct((_S, _D), jnp.float32),
    )(x, wqi, wqo)


def _topk_kernel(q_ref, keys_ref, oidx_ref, rv_ref, ri_ref):
    i = pl.program_id(0)
    nb = pl.num_programs(0)

    @pl.when(i == 0)
    def _():
        rv_ref[...] = jnp.full((_QROWS, 8), _NEG, jnp.float32)
        ri_ref[...] = jnp.full((_QROWS, 8), _IBIG, jnp.int32)

    s = lax.dot_general(
        q_ref[...], keys_ref[...], (((1,), (1,)), ((), ())),
        preferred_element_type=jnp.float32)
    col = lax.broadcasted_iota(jnp.int32, (_QROWS, _KBLK), 1) + i * _KBLK
    lane8 = lax.broadcasted_iota(jnp.int32, (_QROWS, 8), 1)

    bv = jnp.full((_QROWS, 8), _NEG, jnp.float32)
    bi = jnp.full((_QROWS, 8), _IBIG, jnp.int32)
    for t in range(_TOPK):
        v = jnp.max(s, axis=1, keepdims=True)
        idx = jnp.min(jnp.where(s == v, col, _IBIG), axis=1, keepdims=True)
        bv = jnp.where(lane8 == t, v, bv)
        bi = jnp.where(lane8 == t, idx, bi)
        s = jnp.where(col == idx, _NEG, s)

    cv = jnp.concatenate([rv_ref[...], bv], axis=1)
    ci = jnp.concatenate([ri_ref[...], bi], axis=1)
    nv = jnp.full((_QROWS, 8), _NEG, jnp.float32)
    ni = jnp.full((_QROWS, 8), _IBIG, jnp.int32)
    for t in range(_TOPK):
        v = jnp.max(cv, axis=1, keepdims=True)
        idx = jnp.min(jnp.where(cv == v, ci, _IBIG), axis=1, keepdims=True)
        nv = jnp.where(lane8 == t, v, nv)
        ni = jnp.where(lane8 == t, idx, ni)
        cv = jnp.where(ci == idx, _NEG, cv)
    rv_ref[...] = nv
    ri_ref[...] = ni

    @pl.when(i == nb - 1)
    def _():
        oidx_ref[...] = ri_ref[...]


def _retrieval_topk(qa_head, index_keys):
    return pl.pallas_call(
        _topk_kernel,
        grid=(_K // _KBLK,),
        in_specs=[
            pl.BlockSpec((_QROWS, _D), lambda i: (0, 0)),
            pl.BlockSpec((_KBLK, _D), lambda i: (i, 0)),
        ],
        out_specs=pl.BlockSpec((_QROWS, 8), lambda i: (0, 0)),
        out_shape=jax.ShapeDtypeStruct((_QROWS, 8), jnp.int32),
        scratch_shapes=[
            pltpu.VMEM((_QROWS, 8), jnp.float32),
            pltpu.VMEM((_QROWS, 8), jnp.int32),
        ],
    )(qa_head, index_keys)


def _sc_gather(catkv, idx):
    n = idx.shape[0]
    nw = 32
    per = n // nw
    mesh = plsc.VectorSubcoreMesh(core_axis_name="c", subcore_axis_name="s")

    @functools.partial(
        pl.kernel,
        mesh=mesh,
        out_type=jax.ShapeDtypeStruct((n, 2 * _D), jnp.float32),
        scratch_types=[
            pltpu.VMEM((per,), jnp.int32),
            pltpu.VMEM((per, 2 * _D), jnp.float32),
            pltpu.SemaphoreType.DMA,
        ],
    )
    def gk(cat_hbm, idx_hbm, o_hbm, idx_v, rows_v, sem):
        wid = lax.axis_index("s") * 2 + lax.axis_index("c")
        base = wid * per
        pltpu.sync_copy(idx_hbm.at[pl.ds(base, per)], idx_v)
        pltpu.async_copy(cat_hbm.at[idx_v], rows_v, sem).wait()
        pltpu.sync_copy(rows_v, o_hbm.at[pl.ds(base, per)])

    return gk(catkv, idx)


def _flash_kernel(q_ref, k_ref, v_ref, o_ref, acc_ref, m_ref, l_ref,
                  *, scale, bq, bk, hd, split_kv=False):
    qi = pl.program_id(1)
    ki = pl.program_id(2)

    @pl.when(ki == 0)
    def _():
        m_ref[...] = jnp.full((bq, 128), _NEG, jnp.float32)
        l_ref[...] = jnp.zeros((bq, 128), jnp.float32)
        acc_ref[...] = jnp.zeros((bq, hd), jnp.float32)

    def step(masked):
        if split_kv:
            kv = k_ref[...]
            kb = kv[:, :hd]
            vb = kv[:, hd:]
        else:
            kb = k_ref[...]
            vb = v_ref[...]
        s = lax.dot_general(
            q_ref[...].astype(jnp.bfloat16), kb.astype(jnp.bfloat16),
            (((1,), (1,)), ((), ())),
            preferred_element_type=jnp.float32) * scale
        if masked:
            row = lax.broadcasted_iota(jnp.int32, (bq, bk), 0)
            colg = lax.broadcasted_iota(jnp.int32, (bq, bk), 1)
            s = jnp.where(colg <= row, s, _NEG)

        m_prev = m_ref[...][:, :1]
        l_prev = l_ref[...][:, :1]
        m_cur = jnp.max(s, axis=1, keepdims=True)
        m_new = jnp.maximum(m_prev, m_cur)
        alpha = jnp.exp(m_prev - m_new)
        p = jnp.exp(s - m_new)
        l_new = l_prev * alpha + jnp.sum(p, axis=1, keepdims=True)
        acc_ref[...] = acc_ref[...] * alpha + lax.dot_general(
            p.astype(jnp.bfloat16), vb.astype(jnp.bfloat16),
            (((1,), (0,)), ((), ())),
            preferred_element_type=jnp.float32)
        m_ref[...] = jnp.broadcast_to(m_new, (bq, 128))
        l_ref[...] = jnp.broadcast_to(l_new, (bq, 128))
        if masked:
            o_ref[...] = acc_ref[...] / l_ref[...][:, :1]

    @pl.when(ki < qi)
    def _():
        step(False)

    @pl.when(ki == qi)
    def _():
        step(True)


def _flash2_kernel(q_ref, k_ref, v_ref, o_ref, acc_ref, m_ref, l_ref,
                   *, scale, bq, bk):
    qi = pl.program_id(1)
    ki = pl.program_id(2)

    @pl.when(ki == 0)
    def _():
        m_ref[...] = jnp.full((bq, 2 * _HD), _NEG, jnp.float32)
        l_ref[...] = jnp.zeros((bq, 2 * _HD), jnp.float32)
        acc_ref[...] = jnp.zeros((bq, 2 * _HD), jnp.float32)

    def step(masked):
        if masked:
            row = lax.broadcasted_iota(jnp.int32, (bq, bk), 0)
            colg = lax.broadcasted_iota(jnp.int32, (bq, bk), 1)
            keep = colg <= row
        qall = q_ref[...]
        kall = k_ref[...]
        vall = v_ref[...]
        for sub in (0, 1):
            sl = slice(_HD * sub, _HD * (sub + 1))
            qs = (qall[:, sl] * scale).astype(jnp.bfloat16)
            s = lax.dot_general(
                qs, kall[:, sl].astype(jnp.bfloat16),
                (((1,), (1,)), ((), ())),
                preferred_element_type=jnp.float32)
            if masked:
                s = jnp.where(keep, s, _NEG)
            m_prev = m_ref[:, _HD * sub:_HD * sub + 1]
            l_prev = l_ref[:, _HD * sub:_HD * sub + 1]
            m_cur = jnp.max(s, axis=1, keepdims=True)
            m_new = jnp.maximum(m_prev, m_cur)
            alpha = jnp.exp(m_prev - m_new)
            p = jnp.exp(s - m_new)
            l_new = l_prev * alpha + jnp.sum(p, axis=1, keepdims=True)
            acc_ref[:, sl] = acc_ref[:, sl] * alpha + lax.dot_general(
                p.astype(jnp.bfloat16), vall[:, sl].astype(jnp.bfloat16),
                (((1,), (0,)), ((), ())),
                preferred_element_type=jnp.float32)
            m_ref[:, sl] = jnp.broadcast_to(m_new, (bq, _HD))
            l_ref[:, sl] = jnp.broadcast_to(l_new, (bq, _HD))
            if masked:
                o_ref[:, sl] = acc_ref[:, sl] / l_ref[:, _HD * sub:_HD * sub + 1]

    @pl.when(ki < qi)
    def _():
        step(False)

    @pl.when(ki == qi)
    def _():
        step(True)


def _base_attention(q, k, v):
    bq = bk = 512
    nq = _S // bq
    kern = functools.partial(
        _flash2_kernel, scale=1.0 / math.sqrt(_HD), bq=bq, bk=bk)
    spec_q = pl.BlockSpec((bq, 2 * _HD), lambda h, qi, ki: (qi, h))
    spec_kv = pl.BlockSpec((bk, 2 * _HD),
                           lambda h, qi, ki: (jnp.minimum(ki, qi), h))
    return pl.pallas_call(
        kern,
        grid=(_NH // 2, nq, nq),
        in_specs=[spec_q, spec_kv, spec_kv],
        out_specs=pl.BlockSpec((bq, 2 * _HD), lambda h, qi, ki: (qi, h)),
        out_shape=jax.ShapeDtypeStruct((_S, _H), jnp.float32),
        scratch_shapes=[
            pltpu.VMEM((bq, 2 * _HD), jnp.float32),
            pltpu.VMEM((bq, 2 * _HD), jnp.float32),
            pltpu.VMEM((bq, 2 * _HD), jnp.float32),
        ],
    )(q, k, v)


def _adapter_attention(qa, kvg):
    bq = bk = 512
    nq = _S // bq
    kern = functools.partial(
        _flash_kernel, scale=1.0 / math.sqrt(_D), bq=bq, bk=bk, hd=_D,
        split_kv=True)
    return pl.pallas_call(
        kern,
        grid=(1, nq, nq),
        in_specs=[
            pl.BlockSpec((bq, _D), lambda h, qi, ki: (qi, 0)),
            pl.BlockSpec((bk, 2 * _D),
                         lambda h, qi, ki: (jnp.minimum(ki, qi), 0)),
            pl.BlockSpec((bk, 2 * _D),
                         lambda h, qi, ki: (jnp.minimum(ki, qi), 0)),
        ],
        out_specs=pl.BlockSpec((bq, _D), lambda h, qi, ki: (qi, 0)),
        out_shape=jax.ShapeDtypeStruct((_S, _D), jnp.float32),
        scratch_shapes=[
            pltpu.VMEM((bq, _D), jnp.float32),
            pltpu.VMEM((bq, 128), jnp.float32),
            pltpu.VMEM((bq, 128), jnp.float32),
        ],
    )(qa, kvg, kvg)


def _final_kernel(attn_ref, wo_ref, mome_ref, wvi_ref, wvo_ref, o_ref):
    base = lax.dot_general(
        attn_ref[...], wo_ref[...], (((1,), (1,)), ((), ())),
        preferred_element_type=jnp.float32)
    t = lax.dot_general(
        mome_ref[...], wvi_ref[...], (((1,), (1,)), ((), ())),
        preferred_element_type=jnp.float32)
    ad = lax.dot_general(
        t, wvo_ref[...], (((1,), (1,)), ((), ())),
        preferred_element_type=jnp.float32)
    o_ref[...] = base + ad * float(_SCALING)


def _final_proj(attn, wo, mome, wvi, wvo):
    cb = 512
    return pl.pallas_call(
        _final_kernel,
        grid=(_H // cb,),
        in_specs=[
            pl.BlockSpec((_S, _H), lambda i: (0, 0)),
            pl.BlockSpec((cb, _H), lambda i: (i, 0)),
            pl.BlockSpec((_S, _D), lambda i: (0, 0)),
            pl.BlockSpec((_R, _D), lambda i: (0, 0)),
            pl.BlockSpec((cb, _R), lambda i: (i, 0)),
        ],
        out_specs=pl.BlockSpec((_S, cb), lambda i: (0, i)),
        out_shape=jax.ShapeDtypeStruct((_S, _H), jnp.float32),
    )(attn, wo, mome, wvi, wvo)


def kernel(hidden_states, Wq, Wk, Wv, Wo, Wqi, Wqo, Wvi, Wvo,
           index_keys, index_values):
    x = hidden_states.reshape(_S, _H)
    xb = x.astype(jnp.bfloat16)

    q, k, v = _qkv_proj(xb, Wq.astype(jnp.bfloat16), Wk.astype(jnp.bfloat16),
                        Wv.astype(jnp.bfloat16))
    qa = _qa_proj(x, Wqi, Wqo)

    topidx = _retrieval_topk(qa[:_QROWS], index_keys)
    flat_idx = topidx[:410, :_TOPK].reshape(-1)[:_S]

    catkv = jnp.concatenate([index_keys, index_values], axis=1)
    kvg = _sc_gather(catkv, flat_idx)

    attn = _base_attention(q, k, v)
    mome = _adapter_attention(qa, kvg)

    out = _final_proj(attn.astype(jnp.bfloat16), Wo.astype(jnp.bfloat16),
                      mome, Wvi, Wvo)
    return out.reshape(_B, _S, _H)


# speedup vs baseline: 3.8428x; 1.0456x over previous
import functools
import math

import jax
import jax.numpy as jnp
from jax import lax
from jax.experimental import pallas as pl
from jax.experimental.pallas import tpu as pltpu
from jax.experimental.pallas import tpu_sc as plsc

_B, _S, _H = 1, 2048, 2048
_NH = 16
_HD = _H // _NH
_R = 16
_D = 64
_K = 100000
_TOPK = 5
_SCALING = _R

_NEG = -1e30
_IBIG = 2**31 - 1

_QROWS = 416
_KBLK = 2000


def _qkv_kernel(x_ref, wq_ref, wk_ref, wv_ref, q_ref, k_ref, v_ref):
    xb = x_ref[...].astype(jnp.bfloat16)
    for w_ref, o_ref in ((wq_ref, q_ref), (wk_ref, k_ref), (wv_ref, v_ref)):
        o_ref[...] = lax.dot_general(
            xb, w_ref[...].astype(jnp.bfloat16), (((1,), (1,)), ((), ())),
            preferred_element_type=jnp.float32)


def _qkv_proj(x, wq, wk, wv):
    cb = 256
    wspec = pl.BlockSpec((cb, _H), lambda i: (i, 0))
    ospec = pl.BlockSpec((_S, cb), lambda i: (0, i))
    oshape = jax.ShapeDtypeStruct((_S, _H), jnp.float32)
    return pl.pallas_call(
        _qkv_kernel,
        grid=(_H // cb,),
        in_specs=[pl.BlockSpec((_S, _H), lambda i: (0, 0)),
                  wspec, wspec, wspec],
        out_specs=[ospec, ospec, ospec],
        out_shape=[oshape, oshape, oshape],
    )(x, wq, wk, wv)


def _qa_kernel(x_ref, wqi_ref, wqo_ref, o_ref):
    t = lax.dot_general(
        x_ref[...], wqi_ref[...], (((1,), (1,)), ((), ())),
        preferred_element_type=jnp.float32)
    o_ref[...] = lax.dot_general(
        t, wqo_ref[...], (((1,), (1,)), ((), ())),
        preferred_element_type=jnp.float32)


def _qa_proj(x, wqi, wqo):
    bs = 256
    return pl.pallas_call(
        _qa_kernel,
        grid=(_S // bs,),
        in_specs=[
            pl.BlockSpec((bs, _H), lambda i: (i, 0)),
            pl.BlockSpec((_R, _H), lambda i: (0, 0)),
            pl.BlockSpec((_D, _R), lambda i: (0, 0)),
        ],
        out_specs=pl.BlockSpec((bs, _D), lambda i: (i, 0)),
        out_shape=jax.ShapeDtypeStruct((_S, _D), jnp.float32),
    )(x, wqi, wqo)


def _topk_kernel(q_ref, keys_ref, oidx_ref, rv_ref, ri_ref):
    i = pl.program_id(0)
    nb = pl.num_programs(0)

    @pl.when(i == 0)
    def _():
        rv_ref[...] = jnp.full((_QROWS, 8), _NEG, jnp.float32)
        ri_ref[...] = jnp.full((_QROWS, 8), _IBIG, jnp.int32)

    s = lax.dot_general(
        q_ref[...], keys_ref[...], (((1,), (1,)), ((), ())),
        preferred_element_type=jnp.float32)
    col = lax.broadcasted_iota(jnp.int32, (_QROWS, _KBLK), 1) + i * _KBLK
    lane8 = lax.broadcasted_iota(jnp.int32, (_QROWS, 8), 1)

    bv = jnp.full((_QROWS, 8), _NEG, jnp.float32)
    bi = jnp.full((_QROWS, 8), _IBIG, jnp.int32)
    for t in range(_TOPK):
        v = jnp.max(s, axis=1, keepdims=True)
        idx = jnp.min(jnp.where(s == v, col, _IBIG), axis=1, keepdims=True)
        bv = jnp.where(lane8 == t, v, bv)
        bi = jnp.where(lane8 == t, idx, bi)
        s = jnp.where(col == idx, _NEG, s)

    cv = jnp.concatenate([rv_ref[...], bv], axis=1)
    ci = jnp.concatenate([ri_ref[...], bi], axis=1)
    nv = jnp.full((_QROWS, 8), _NEG, jnp.float32)
    ni = jnp.full((_QROWS, 8), _IBIG, jnp.int32)
    for t in range(_TOPK):
        v = jnp.max(cv, axis=1, keepdims=True)
        idx = jnp.min(jnp.where(cv == v, ci, _IBIG), axis=1, keepdims=True)
        nv = jnp.where(lane8 == t, v, nv)
        ni = jnp.where(lane8 == t, idx, ni)
        cv = jnp.where(ci == idx, _NEG, cv)
    rv_ref[...] = nv
    ri_ref[...] = ni

    @pl.when(i == nb - 1)
    def _():
        oidx_ref[...] = ri_ref[...]


def _retrieval_topk(qa_head, index_keys):
    return pl.pallas_call(
        _topk_kernel,
        grid=(_K // _KBLK,),
        in_specs=[
            pl.BlockSpec((_QROWS, _D), lambda i: (0, 0)),
            pl.BlockSpec((_KBLK, _D), lambda i: (i, 0)),
        ],
        out_specs=pl.BlockSpec((_QROWS, 8), lambda i: (0, 0)),
        out_shape=jax.ShapeDtypeStruct((_QROWS, 8), jnp.int32),
        scratch_shapes=[
            pltpu.VMEM((_QROWS, 8), jnp.float32),
            pltpu.VMEM((_QROWS, 8), jnp.int32),
        ],
    )(qa_head, index_keys)


def _catkv_kernel(k_ref, v_ref, o_ref):
    o_ref[:, :_D] = k_ref[...]
    o_ref[:, _D:] = v_ref[...]


def _catkv_build(keys, values):
    blk = 4000
    return pl.pallas_call(
        _catkv_kernel,
        grid=(_K // blk,),
        in_specs=[
            pl.BlockSpec((blk, _D), lambda i: (i, 0)),
            pl.BlockSpec((blk, _D), lambda i: (i, 0)),
        ],
        out_specs=pl.BlockSpec((blk, 2 * _D), lambda i: (i, 0)),
        out_shape=jax.ShapeDtypeStruct((_K, 2 * _D), jnp.float32),
    )(keys, values)


def _sc_gather(catkv, idx):
    n = idx.shape[0]
    nw = 32
    per = n // nw
    mesh = plsc.VectorSubcoreMesh(core_axis_name="c", subcore_axis_name="s")

    @functools.partial(
        pl.kernel,
        mesh=mesh,
        out_type=jax.ShapeDtypeStruct((n, 2 * _D), jnp.float32),
        scratch_types=[
            pltpu.VMEM((per,), jnp.int32),
            pltpu.VMEM((per, 2 * _D), jnp.float32),
            pltpu.SemaphoreType.DMA,
        ],
    )
    def gk(cat_hbm, idx_hbm, o_hbm, idx_v, rows_v, sem):
        wid = lax.axis_index("s") * 2 + lax.axis_index("c")
        base = wid * per
        pltpu.sync_copy(idx_hbm.at[pl.ds(base, per)], idx_v)
        pltpu.async_copy(cat_hbm.at[idx_v], rows_v, sem).wait()
        pltpu.sync_copy(rows_v, o_hbm.at[pl.ds(base, per)])

    return gk(catkv, idx)


def _flash_kernel(q_ref, k_ref, v_ref, o_ref, acc_ref, m_ref, l_ref,
                  *, scale, bq, bk, hd, split_kv=False):
    qi = pl.program_id(1)
    ki = pl.program_id(2)

    @pl.when(ki == 0)
    def _():
        m_ref[...] = jnp.full((bq, 128), _NEG, jnp.float32)
        l_ref[...] = jnp.zeros((bq, 128), jnp.float32)
        acc_ref[...] = jnp.zeros((bq, hd), jnp.float32)

    def step(masked):
        if split_kv:
            kv = k_ref[...]
            kb = kv[:, :hd]
            vb = kv[:, hd:]
        else:
            kb = k_ref[...]
            vb = v_ref[...]
        s = lax.dot_general(
            q_ref[...].astype(jnp.bfloat16), kb.astype(jnp.bfloat16),
            (((1,), (1,)), ((), ())),
            preferred_element_type=jnp.float32) * scale
        if masked:
            row = lax.broadcasted_iota(jnp.int32, (bq, bk), 0)
            colg = lax.broadcasted_iota(jnp.int32, (bq, bk), 1)
            s = jnp.where(colg <= row, s, _NEG)

        m_prev = m_ref[...][:, :1]
        l_prev = l_ref[...][:, :1]
        m_cur = jnp.max(s, axis=1, keepdims=True)
        m_new = jnp.maximum(m_prev, m_cur)
        alpha = jnp.exp(m_prev - m_new)
        p = jnp.exp(s - m_new)
        l_new = l_prev * alpha + jnp.sum(p, axis=1, keepdims=True)
        acc_ref[...] = acc_ref[...] * alpha + lax.dot_general(
            p.astype(jnp.bfloat16), vb.astype(jnp.bfloat16),
            (((1,), (0,)), ((), ())),
            preferred_element_type=jnp.float32)
        m_ref[...] = jnp.broadcast_to(m_new, (bq, 128))
        l_ref[...] = jnp.broadcast_to(l_new, (bq, 128))
        if masked:
            o_ref[...] = acc_ref[...] / l_ref[...][:, :1]

    @pl.when(ki < qi)
    def _():
        step(False)

    @pl.when(ki == qi)
    def _():
        step(True)


def _flash2_kernel(q_ref, k_ref, v_ref, o_ref, acc_ref, m_ref, l_ref,
                   *, scale, bq, bk, hp):
    qi = pl.program_id(1)
    ki = pl.program_id(2)

    @pl.when(ki == 0)
    def _():
        m_ref[...] = jnp.full((bq, hp * _HD), _NEG, jnp.float32)
        l_ref[...] = jnp.zeros((bq, hp * _HD), jnp.float32)
        acc_ref[...] = jnp.zeros((bq, hp * _HD), jnp.float32)

    def step(masked):
        if masked:
            row = lax.broadcasted_iota(jnp.int32, (bq, bk), 0)
            colg = lax.broadcasted_iota(jnp.int32, (bq, bk), 1)
            keep = colg <= row
        qall = q_ref[...]
        kall = k_ref[...]
        vall = v_ref[...]
        for sub in range(hp):
            sl = slice(_HD * sub, _HD * (sub + 1))
            qs = (qall[:, sl] * scale).astype(jnp.bfloat16)
            s = lax.dot_general(
                qs, kall[:, sl].astype(jnp.bfloat16),
                (((1,), (1,)), ((), ())),
                preferred_element_type=jnp.float32)
            if masked:
                s = jnp.where(keep, s, _NEG)
            m_prev = m_ref[:, _HD * sub:_HD * sub + 1]
            l_prev = l_ref[:, _HD * sub:_HD * sub + 1]
            m_cur = jnp.max(s, axis=1, keepdims=True)
            m_new = jnp.maximum(m_prev, m_cur)
            alpha = jnp.exp(m_prev - m_new)
            p = jnp.exp(s - m_new)
            l_new = l_prev * alpha + jnp.sum(p, axis=1, keepdims=True)
            acc_ref[:, sl] = acc_ref[:, sl] * alpha + lax.dot_general(
                p.astype(jnp.bfloat16), vall[:, sl].astype(jnp.bfloat16),
                (((1,), (0,)), ((), ())),
                preferred_element_type=jnp.float32)
            m_ref[:, sl] = jnp.broadcast_to(m_new, (bq, _HD))
            l_ref[:, sl] = jnp.broadcast_to(l_new, (bq, _HD))
            if masked:
                o_ref[:, sl] = acc_ref[:, sl] / l_ref[:, _HD * sub:_HD * sub + 1]

    @pl.when(ki < qi)
    def _():
        step(False)

    @pl.when(ki == qi)
    def _():
        step(True)


def _base_attention(q, k, v):
    bq = bk = 512
    hp = 4
    nq = _S // bq
    kern = functools.partial(
        _flash2_kernel, scale=1.0 / math.sqrt(_HD), bq=bq, bk=bk, hp=hp)
    spec_q = pl.BlockSpec((bq, hp * _HD), lambda h, qi, ki: (qi, h))
    spec_kv = pl.BlockSpec((bk, hp * _HD),
                           lambda h, qi, ki: (jnp.minimum(ki, qi), h))
    return pl.pallas_call(
        kern,
        grid=(_NH // hp, nq, nq),
        in_specs=[spec_q, spec_kv, spec_kv],
        out_specs=pl.BlockSpec((bq, hp * _HD), lambda h, qi, ki: (qi, h)),
        out_shape=jax.ShapeDtypeStruct((_S, _H), jnp.float32),
        scratch_shapes=[
            pltpu.VMEM((bq, hp * _HD), jnp.float32),
            pltpu.VMEM((bq, hp * _HD), jnp.float32),
            pltpu.VMEM((bq, hp * _HD), jnp.float32),
        ],
    )(q, k, v)


def _adapter_attention(qa, kvg):
    bq = bk = 512
    nq = _S // bq
    kern = functools.partial(
        _flash_kernel, scale=1.0 / math.sqrt(_D), bq=bq, bk=bk, hd=_D,
        split_kv=True)
    return pl.pallas_call(
        kern,
        grid=(1, nq, nq),
        in_specs=[
            pl.BlockSpec((bq, _D), lambda h, qi, ki: (qi, 0)),
            pl.BlockSpec((bk, 2 * _D),
                         lambda h, qi, ki: (jnp.minimum(ki, qi), 0)),
            pl.BlockSpec((bk, 2 * _D),
                         lambda h, qi, ki: (jnp.minimum(ki, qi), 0)),
        ],
        out_specs=pl.BlockSpec((bq, _D), lambda h, qi, ki: (qi, 0)),
        out_shape=jax.ShapeDtypeStruct((_S, _D), jnp.float32),
        scratch_shapes=[
            pltpu.VMEM((bq, _D), jnp.float32),
            pltpu.VMEM((bq, 128), jnp.float32),
            pltpu.VMEM((bq, 128), jnp.float32),
        ],
    )(qa, kvg, kvg)


def _final_kernel(attn_ref, wo_ref, mome_ref, wvi_ref, wvo_ref, o_ref):
    base = lax.dot_general(
        attn_ref[...].astype(jnp.bfloat16), wo_ref[...].astype(jnp.bfloat16),
        (((1,), (1,)), ((), ())),
        preferred_element_type=jnp.float32)
    t = lax.dot_general(
        mome_ref[...], wvi_ref[...], (((1,), (1,)), ((), ())),
        preferred_element_type=jnp.float32)
    ad = lax.dot_general(
        t, wvo_ref[...], (((1,), (1,)), ((), ())),
        preferred_element_type=jnp.float32)
    o_ref[...] = base + ad * float(_SCALING)


def _final_proj(attn, wo, mome, wvi, wvo):
    cb = 512
    return pl.pallas_call(
        _final_kernel,
        grid=(_H // cb,),
        in_specs=[
            pl.BlockSpec((_S, _H), lambda i: (0, 0)),
            pl.BlockSpec((cb, _H), lambda i: (i, 0)),
            pl.BlockSpec((_S, _D), lambda i: (0, 0)),
            pl.BlockSpec((_R, _D), lambda i: (0, 0)),
            pl.BlockSpec((cb, _R), lambda i: (i, 0)),
        ],
        out_specs=pl.BlockSpec((_S, cb), lambda i: (0, i)),
        out_shape=jax.ShapeDtypeStruct((_S, _H), jnp.float32),
    )(attn, wo, mome, wvi, wvo)


def kernel(hidden_states, Wq, Wk, Wv, Wo, Wqi, Wqo, Wvi, Wvo,
           index_keys, index_values):
    x = hidden_states.reshape(_S, _H)

    q, k, v = _qkv_proj(x, Wq, Wk, Wv)
    qa = _qa_proj(x, Wqi, Wqo)

    topidx = _retrieval_topk(qa[:_QROWS], index_keys)
    flat_idx = topidx[:410, :_TOPK].reshape(-1)[:_S]

    catkv = _catkv_build(index_keys, index_values)
    kvg = _sc_gather(catkv, flat_idx)

    attn = _base_attention(q, k, v)
    mome = _adapter_attention(qa, kvg)

    out = _final_proj(attn, Wo, mome, Wvi, Wvo)
    return out.reshape(_B, _S, _H)


# speedup vs baseline: 4.2046x; 1.0941x over previous
import functools
import math

import jax
import jax.numpy as jnp
from jax import lax
from jax.experimental import pallas as pl
from jax.experimental.pallas import tpu as pltpu
from jax.experimental.pallas import tpu_sc as plsc

_B, _S, _H = 1, 2048, 2048
_NH = 16
_HD = _H // _NH
_R = 16
_D = 64
_K = 100000
_TOPK = 5
_SCALING = _R

_NEG = -1e30
_IBIG = 2**31 - 1

_QROWS = 416
_KBLK = 2000


def _qkv_kernel(x_ref, wq_ref, wk_ref, wv_ref, q_ref, k_ref, v_ref):
    xb = x_ref[...].astype(jnp.bfloat16)
    for w_ref, o_ref in ((wq_ref, q_ref), (wk_ref, k_ref), (wv_ref, v_ref)):
        o_ref[...] = lax.dot_general(
            xb, w_ref[...].astype(jnp.bfloat16), (((1,), (1,)), ((), ())),
            preferred_element_type=jnp.float32)


def _qkv_proj(x, wq, wk, wv):
    cb = 256
    wspec = pl.BlockSpec((cb, _H), lambda i: (i, 0))
    ospec = pl.BlockSpec((_S, cb), lambda i: (0, i))
    oshape = jax.ShapeDtypeStruct((_S, _H), jnp.float32)
    return pl.pallas_call(
        _qkv_kernel,
        grid=(_H // cb,),
        in_specs=[pl.BlockSpec((_S, _H), lambda i: (0, 0)),
                  wspec, wspec, wspec],
        out_specs=[ospec, ospec, ospec],
        out_shape=[oshape, oshape, oshape],
    )(x, wq, wk, wv)


def _qa_kernel(x_ref, wqi_ref, wqo_ref, o_ref):
    t = lax.dot_general(
        x_ref[...], wqi_ref[...], (((1,), (1,)), ((), ())),
        preferred_element_type=jnp.float32)
    o_ref[...] = lax.dot_general(
        t, wqo_ref[...], (((1,), (1,)), ((), ())),
        preferred_element_type=jnp.float32)


def _qa_proj(x, wqi, wqo):
    bs = 256
    return pl.pallas_call(
        _qa_kernel,
        grid=(_S // bs,),
        in_specs=[
            pl.BlockSpec((bs, _H), lambda i: (i, 0)),
            pl.BlockSpec((_R, _H), lambda i: (0, 0)),
            pl.BlockSpec((_D, _R), lambda i: (0, 0)),
        ],
        out_specs=pl.BlockSpec((bs, _D), lambda i: (i, 0)),
        out_shape=jax.ShapeDtypeStruct((_S, _D), jnp.float32),
    )(x, wqi, wqo)


def _topk_kernel(q_ref, keys_ref, oidx_ref, rv_ref, ri_ref):
    i = pl.program_id(0)
    nb = pl.num_programs(0)

    @pl.when(i == 0)
    def _():
        rv_ref[...] = jnp.full((_QROWS, 8), _NEG, jnp.float32)
        ri_ref[...] = jnp.full((_QROWS, 8), _IBIG, jnp.int32)

    s = lax.dot_general(
        q_ref[...], keys_ref[...][:, :_D], (((1,), (1,)), ((), ())),
        preferred_element_type=jnp.float32)
    col = lax.broadcasted_iota(jnp.int32, (_QROWS, _KBLK), 1) + i * _KBLK
    lane8 = lax.broadcasted_iota(jnp.int32, (_QROWS, 8), 1)

    bv = jnp.full((_QROWS, 8), _NEG, jnp.float32)
    bi = jnp.full((_QROWS, 8), _IBIG, jnp.int32)
    for t in range(_TOPK):
        v = jnp.max(s, axis=1, keepdims=True)
        idx = jnp.min(jnp.where(s == v, col, _IBIG), axis=1, keepdims=True)
        bv = jnp.where(lane8 == t, v, bv)
        bi = jnp.where(lane8 == t, idx, bi)
        s = jnp.where(col == idx, _NEG, s)

    cv = jnp.concatenate([rv_ref[...], bv], axis=1)
    ci = jnp.concatenate([ri_ref[...], bi], axis=1)
    nv = jnp.full((_QROWS, 8), _NEG, jnp.float32)
    ni = jnp.full((_QROWS, 8), _IBIG, jnp.int32)
    for t in range(_TOPK):
        v = jnp.max(cv, axis=1, keepdims=True)
        idx = jnp.min(jnp.where(cv == v, ci, _IBIG), axis=1, keepdims=True)
        nv = jnp.where(lane8 == t, v, nv)
        ni = jnp.where(lane8 == t, idx, ni)
        cv = jnp.where(ci == idx, _NEG, cv)
    rv_ref[...] = nv
    ri_ref[...] = ni

    @pl.when(i == nb - 1)
    def _():
        oidx_ref[...] = ri_ref[...]


def _retrieval_topk(qa_head, index_keys):
    return pl.pallas_call(
        _topk_kernel,
        grid=(_K // _KBLK,),
        in_specs=[
            pl.BlockSpec((_QROWS, _D), lambda i: (0, 0)),
            pl.BlockSpec((_KBLK, 2 * _D), lambda i: (i, 0)),
        ],
        out_specs=pl.BlockSpec((_QROWS, 8), lambda i: (0, 0)),
        out_shape=jax.ShapeDtypeStruct((_QROWS, 8), jnp.int32),
        scratch_shapes=[
            pltpu.VMEM((_QROWS, 8), jnp.float32),
            pltpu.VMEM((_QROWS, 8), jnp.int32),
        ],
    )(qa_head, index_keys)


def _catkv_kernel(k_ref, v_ref, o_ref):
    o_ref[:, :_D] = k_ref[...]
    o_ref[:, _D:] = v_ref[...]


def _catkv_build(keys, values):
    blk = 4000
    return pl.pallas_call(
        _catkv_kernel,
        grid=(_K // blk,),
        in_specs=[
            pl.BlockSpec((blk, _D), lambda i: (i, 0)),
            pl.BlockSpec((blk, _D), lambda i: (i, 0)),
        ],
        out_specs=pl.BlockSpec((blk, 2 * _D), lambda i: (i, 0)),
        out_shape=jax.ShapeDtypeStruct((_K, 2 * _D), jnp.float32),
    )(keys, values)


def _sc_gather(catkv, idx):
    n = idx.shape[0]
    nw = 32
    per = n // nw
    mesh = plsc.VectorSubcoreMesh(core_axis_name="c", subcore_axis_name="s")

    @functools.partial(
        pl.kernel,
        mesh=mesh,
        out_type=jax.ShapeDtypeStruct((n, 2 * _D), jnp.float32),
        scratch_types=[
            pltpu.VMEM((per,), jnp.int32),
            pltpu.VMEM((per, 2 * _D), jnp.float32),
            pltpu.SemaphoreType.DMA,
        ],
    )
    def gk(cat_hbm, idx_hbm, o_hbm, idx_v, rows_v, sem):
        wid = lax.axis_index("s") * 2 + lax.axis_index("c")
        base = wid * per
        pltpu.sync_copy(idx_hbm.at[pl.ds(base, per)], idx_v)
        pltpu.async_copy(cat_hbm.at[idx_v], rows_v, sem).wait()
        pltpu.sync_copy(rows_v, o_hbm.at[pl.ds(base, per)])

    return gk(catkv, idx)


def _flash_kernel(q_ref, k_ref, v_ref, o_ref, acc_ref, m_ref, l_ref,
                  *, scale, bq, bk, hd, split_kv=False):
    qi = pl.program_id(1)
    ki = pl.program_id(2)

    @pl.when(ki == 0)
    def _():
        m_ref[...] = jnp.full((bq, 128), _NEG, jnp.float32)
        l_ref[...] = jnp.zeros((bq, 128), jnp.float32)
        acc_ref[...] = jnp.zeros((bq, hd), jnp.float32)

    def step(masked):
        if split_kv:
            kv = k_ref[...]
            kb = kv[:, :hd]
            vb = kv[:, hd:]
        else:
            kb = k_ref[...]
            vb = v_ref[...]
        s = lax.dot_general(
            q_ref[...].astype(jnp.bfloat16), kb.astype(jnp.bfloat16),
            (((1,), (1,)), ((), ())),
            preferred_element_type=jnp.float32) * scale
        if masked:
            row = lax.broadcasted_iota(jnp.int32, (bq, bk), 0)
            colg = lax.broadcasted_iota(jnp.int32, (bq, bk), 1)
            s = jnp.where(colg <= row, s, _NEG)

        m_prev = m_ref[...][:, :1]
        l_prev = l_ref[...][:, :1]
        m_cur = jnp.max(s, axis=1, keepdims=True)
        m_new = jnp.maximum(m_prev, m_cur)
        alpha = jnp.exp(m_prev - m_new)
        p = jnp.exp(s - m_new)
        l_new = l_prev * alpha + jnp.sum(p, axis=1, keepdims=True)
        acc_ref[...] = acc_ref[...] * alpha + lax.dot_general(
            p.astype(jnp.bfloat16), vb.astype(jnp.bfloat16),
            (((1,), (0,)), ((), ())),
            preferred_element_type=jnp.float32)
        m_ref[...] = jnp.broadcast_to(m_new, (bq, 128))
        l_ref[...] = jnp.broadcast_to(l_new, (bq, 128))
        if masked:
            o_ref[...] = acc_ref[...] / l_ref[...][:, :1]

    @pl.when(ki < qi)
    def _():
        step(False)

    @pl.when(ki == qi)
    def _():
        step(True)


def _flash2_kernel(q_ref, k_ref, v_ref, o_ref, acc_ref, m_ref, l_ref,
                   *, scale, bq, bk, hp):
    qi = pl.program_id(1)
    ki = pl.program_id(2)

    @pl.when(ki == 0)
    def _():
        m_ref[...] = jnp.full((bq, hp * _HD), _NEG, jnp.float32)
        l_ref[...] = jnp.zeros((bq, hp * _HD), jnp.float32)
        acc_ref[...] = jnp.zeros((bq, hp * _HD), jnp.float32)

    def step(masked):
        if masked:
            row = lax.broadcasted_iota(jnp.int32, (bq, bk), 0)
            colg = lax.broadcasted_iota(jnp.int32, (bq, bk), 1)
            keep = colg <= row
        qall = q_ref[...]
        kall = k_ref[...]
        vall = v_ref[...]
        for sub in range(hp):
            sl = slice(_HD * sub, _HD * (sub + 1))
            qs = (qall[:, sl] * scale).astype(jnp.bfloat16)
            s = lax.dot_general(
                qs, kall[:, sl].astype(jnp.bfloat16),
                (((1,), (1,)), ((), ())),
                preferred_element_type=jnp.float32)
            if masked:
                s = jnp.where(keep, s, _NEG)
            m_prev = m_ref[:, _HD * sub:_HD * sub + 1]
            l_prev = l_ref[:, _HD * sub:_HD * sub + 1]
            m_cur = jnp.max(s, axis=1, keepdims=True)
            m_new = jnp.maximum(m_prev, m_cur)
            alpha = jnp.exp(m_prev - m_new)
            p = jnp.exp(s - m_new)
            l_new = l_prev * alpha + jnp.sum(p, axis=1, keepdims=True)
            acc_ref[:, sl] = acc_ref[:, sl] * alpha + lax.dot_general(
                p.astype(jnp.bfloat16), vall[:, sl].astype(jnp.bfloat16),
                (((1,), (0,)), ((), ())),
                preferred_element_type=jnp.float32)
            m_ref[:, sl] = jnp.broadcast_to(m_new, (bq, _HD))
            l_ref[:, sl] = jnp.broadcast_to(l_new, (bq, _HD))
            if masked:
                o_ref[:, sl] = acc_ref[:, sl] / l_ref[:, _HD * sub:_HD * sub + 1]

    @pl.when(ki < qi)
    def _():
        step(False)

    @pl.when(ki == qi)
    def _():
        step(True)


def _base_attention(q, k, v):
    bq = bk = 512
    hp = 8
    nq = _S // bq
    kern = functools.partial(
        _flash2_kernel, scale=1.0 / math.sqrt(_HD), bq=bq, bk=bk, hp=hp)
    spec_q = pl.BlockSpec((bq, hp * _HD), lambda h, qi, ki: (qi, h))
    spec_kv = pl.BlockSpec((bk, hp * _HD),
                           lambda h, qi, ki: (jnp.minimum(ki, qi), h))
    return pl.pallas_call(
        kern,
        grid=(_NH // hp, nq, nq),
        in_specs=[spec_q, spec_kv, spec_kv],
        out_specs=pl.BlockSpec((bq, hp * _HD), lambda h, qi, ki: (qi, h)),
        out_shape=jax.ShapeDtypeStruct((_S, _H), jnp.float32),
        scratch_shapes=[
            pltpu.VMEM((bq, hp * _HD), jnp.float32),
            pltpu.VMEM((bq, hp * _HD), jnp.float32),
            pltpu.VMEM((bq, hp * _HD), jnp.float32),
        ],
    )(q, k, v)


def _adapter_attention(qa, kvg):
    bq = bk = 512
    nq = _S // bq
    kern = functools.partial(
        _flash_kernel, scale=1.0 / math.sqrt(_D), bq=bq, bk=bk, hd=_D,
        split_kv=True)
    return pl.pallas_call(
        kern,
        grid=(1, nq, nq),
        in_specs=[
            pl.BlockSpec((bq, _D), lambda h, qi, ki: (qi, 0)),
            pl.BlockSpec((bk, 2 * _D),
                         lambda h, qi, ki: (jnp.minimum(ki, qi), 0)),
            pl.BlockSpec((bk, 2 * _D),
                         lambda h, qi, ki: (jnp.minimum(ki, qi), 0)),
        ],
        out_specs=pl.BlockSpec((bq, _D), lambda h, qi, ki: (qi, 0)),
        out_shape=jax.ShapeDtypeStruct((_S, _D), jnp.float32),
        scratch_shapes=[
            pltpu.VMEM((bq, _D), jnp.float32),
            pltpu.VMEM((bq, 128), jnp.float32),
            pltpu.VMEM((bq, 128), jnp.float32),
        ],
    )(qa, kvg, kvg)


def _final_kernel(attn_ref, wo_ref, mome_ref, wvi_ref, wvo_ref, o_ref):
    base = lax.dot_general(
        attn_ref[...].astype(jnp.bfloat16), wo_ref[...].astype(jnp.bfloat16),
        (((1,), (1,)), ((), ())),
        preferred_element_type=jnp.float32)
    t = lax.dot_general(
        mome_ref[...], wvi_ref[...], (((1,), (1,)), ((), ())),
        preferred_element_type=jnp.float32)
    ad = lax.dot_general(
        t, wvo_ref[...], (((1,), (1,)), ((), ())),
        preferred_element_type=jnp.float32)
    o_ref[...] = base + ad * float(_SCALING)


def _final_proj(attn, wo, mome, wvi, wvo):
    cb = 512
    return pl.pallas_call(
        _final_kernel,
        grid=(_H // cb,),
        in_specs=[
            pl.BlockSpec((_S, _H), lambda i: (0, 0)),
            pl.BlockSpec((cb, _H), lambda i: (i, 0)),
            pl.BlockSpec((_S, _D), lambda i: (0, 0)),
            pl.BlockSpec((_R, _D), lambda i: (0, 0)),
            pl.BlockSpec((cb, _R), lambda i: (i, 0)),
        ],
        out_specs=pl.BlockSpec((_S, cb), lambda i: (0, i)),
        out_shape=jax.ShapeDtypeStruct((_S, _H), jnp.float32),
    )(attn, wo, mome, wvi, wvo)


def kernel(hidden_states, Wq, Wk, Wv, Wo, Wqi, Wqo, Wvi, Wvo,
           index_keys, index_values):
    x = hidden_states.reshape(_S, _H)

    q, k, v = _qkv_proj(x, Wq, Wk, Wv)
    qa = _qa_proj(x, Wqi, Wqo)

    catkv = jnp.concatenate([index_keys, index_values], axis=1)
    topidx = _retrieval_topk(qa[:_QROWS], catkv)
    flat_idx = topidx[:410, :_TOPK].reshape(-1)[:_S]

    kvg = _sc_gather(catkv, flat_idx)

    attn = _base_attention(q, k, v)
    mome = _adapter_attention(qa, kvg)

    out = _final_proj(attn, Wo, mome, Wvi, Wvo)
    return out.reshape(_B, _S, _H)


# speedup vs baseline: 4.2386x; 1.0081x over previous
import functools
import math

import jax
import jax.numpy as jnp
from jax import lax
from jax.experimental import pallas as pl
from jax.experimental.pallas import tpu as pltpu
from jax.experimental.pallas import tpu_sc as plsc

_B, _S, _H = 1, 2048, 2048
_NH = 16
_HD = _H // _NH
_R = 16
_D = 64
_K = 100000
_TOPK = 5
_SCALING = _R

_NEG = -1e30
_IBIG = 2**31 - 1

_QROWS = 416
_KBLK = 2000


def _qkv_kernel(x_ref, wq_ref, wk_ref, wv_ref, q_ref, k_ref, v_ref):
    xb = x_ref[...].astype(jnp.bfloat16)
    qscale = 1.0 / math.sqrt(_HD)
    for w_ref, o_ref, sc in ((wq_ref, q_ref, qscale), (wk_ref, k_ref, None),
                             (wv_ref, v_ref, None)):
        acc = lax.dot_general(
            xb, w_ref[...].astype(jnp.bfloat16), (((1,), (1,)), ((), ())),
            preferred_element_type=jnp.float32)
        if sc is not None:
            acc = acc * sc
        o_ref[...] = acc.astype(jnp.bfloat16)


def _qkv_proj(x, wq, wk, wv):
    cb = 256
    wspec = pl.BlockSpec((cb, _H), lambda i: (i, 0))
    ospec = pl.BlockSpec((_S, cb), lambda i: (0, i))
    oshape = jax.ShapeDtypeStruct((_S, _H), jnp.bfloat16)
    return pl.pallas_call(
        _qkv_kernel,
        grid=(_H // cb,),
        in_specs=[pl.BlockSpec((_S, _H), lambda i: (0, 0)),
                  wspec, wspec, wspec],
        out_specs=[ospec, ospec, ospec],
        out_shape=[oshape, oshape, oshape],
    )(x, wq, wk, wv)


def _qa_kernel(x_ref, wqi_ref, wqo_ref, o_ref):
    t = lax.dot_general(
        x_ref[...], wqi_ref[...], (((1,), (1,)), ((), ())),
        preferred_element_type=jnp.float32)
    o_ref[...] = lax.dot_general(
        t, wqo_ref[...], (((1,), (1,)), ((), ())),
        preferred_element_type=jnp.float32)


def _qa_proj(x, wqi, wqo):
    bs = 256
    return pl.pallas_call(
        _qa_kernel,
        grid=(_S // bs,),
        in_specs=[
            pl.BlockSpec((bs, _H), lambda i: (i, 0)),
            pl.BlockSpec((_R, _H), lambda i: (0, 0)),
            pl.BlockSpec((_D, _R), lambda i: (0, 0)),
        ],
        out_specs=pl.BlockSpec((bs, _D), lambda i: (i, 0)),
        out_shape=jax.ShapeDtypeStruct((_S, _D), jnp.float32),
    )(x, wqi, wqo)


def _topk_kernel(q_ref, keys_ref, oidx_ref, rv_ref, ri_ref):
    i = pl.program_id(0)
    nb = pl.num_programs(0)

    @pl.when(i == 0)
    def _():
        rv_ref[...] = jnp.full((_QROWS, 8), _NEG, jnp.float32)
        ri_ref[...] = jnp.full((_QROWS, 8), _IBIG, jnp.int32)

    s = lax.dot_general(
        q_ref[...], keys_ref[...][:, :_D], (((1,), (1,)), ((), ())),
        preferred_element_type=jnp.float32)
    col = lax.broadcasted_iota(jnp.int32, (_QROWS, _KBLK), 1) + i * _KBLK
    lane8 = lax.broadcasted_iota(jnp.int32, (_QROWS, 8), 1)

    bv = jnp.full((_QROWS, 8), _NEG, jnp.float32)
    bi = jnp.full((_QROWS, 8), _IBIG, jnp.int32)
    for t in range(_TOPK):
        v = jnp.max(s, axis=1, keepdims=True)
        idx = jnp.min(jnp.where(s == v, col, _IBIG), axis=1, keepdims=True)
        bv = jnp.where(lane8 == t, v, bv)
        bi = jnp.where(lane8 == t, idx, bi)
        s = jnp.where(col == idx, _NEG, s)

    cv = jnp.concatenate([rv_ref[...], bv], axis=1)
    ci = jnp.concatenate([ri_ref[...], bi], axis=1)
    nv = jnp.full((_QROWS, 8), _NEG, jnp.float32)
    ni = jnp.full((_QROWS, 8), _IBIG, jnp.int32)
    for t in range(_TOPK):
        v = jnp.max(cv, axis=1, keepdims=True)
        idx = jnp.min(jnp.where(cv == v, ci, _IBIG), axis=1, keepdims=True)
        nv = jnp.where(lane8 == t, v, nv)
        ni = jnp.where(lane8 == t, idx, ni)
        cv = jnp.where(ci == idx, _NEG, cv)
    rv_ref[...] = nv
    ri_ref[...] = ni

    @pl.when(i == nb - 1)
    def _():
        oidx_ref[...] = ri_ref[...]


def _retrieval_topk(qa_head, index_keys):
    return pl.pallas_call(
        _topk_kernel,
        grid=(_K // _KBLK,),
        in_specs=[
            pl.BlockSpec((_QROWS, _D), lambda i: (0, 0)),
            pl.BlockSpec((_KBLK, 2 * _D), lambda i: (i, 0)),
        ],
        out_specs=pl.BlockSpec((_QROWS, 8), lambda i: (0, 0)),
        out_shape=jax.ShapeDtypeStruct((_QROWS, 8), jnp.int32),
        scratch_shapes=[
            pltpu.VMEM((_QROWS, 8), jnp.float32),
            pltpu.VMEM((_QROWS, 8), jnp.int32),
        ],
    )(qa_head, index_keys)


def _catkv_kernel(k_ref, v_ref, o_ref):
    o_ref[:, :_D] = k_ref[...]
    o_ref[:, _D:] = v_ref[...]


def _catkv_build(keys, values):
    blk = 4000
    return pl.pallas_call(
        _catkv_kernel,
        grid=(_K // blk,),
        in_specs=[
            pl.BlockSpec((blk, _D), lambda i: (i, 0)),
            pl.BlockSpec((blk, _D), lambda i: (i, 0)),
        ],
        out_specs=pl.BlockSpec((blk, 2 * _D), lambda i: (i, 0)),
        out_shape=jax.ShapeDtypeStruct((_K, 2 * _D), jnp.float32),
    )(keys, values)


def _sc_gather(catkv, idx):
    n = idx.shape[0]
    nw = 32
    per = n // nw
    mesh = plsc.VectorSubcoreMesh(core_axis_name="c", subcore_axis_name="s")

    @functools.partial(
        pl.kernel,
        mesh=mesh,
        out_type=jax.ShapeDtypeStruct((n, 2 * _D), jnp.float32),
        scratch_types=[
            pltpu.VMEM((per,), jnp.int32),
            pltpu.VMEM((per, 2 * _D), jnp.float32),
            pltpu.SemaphoreType.DMA,
        ],
    )
    def gk(cat_hbm, idx_hbm, o_hbm, idx_v, rows_v, sem):
        wid = lax.axis_index("s") * 2 + lax.axis_index("c")
        base = wid * per
        pltpu.sync_copy(idx_hbm.at[pl.ds(base, per)], idx_v)
        pltpu.async_copy(cat_hbm.at[idx_v], rows_v, sem).wait()
        pltpu.sync_copy(rows_v, o_hbm.at[pl.ds(base, per)])

    return gk(catkv, idx)


def _flash_kernel(q_ref, k_ref, v_ref, o_ref, acc_ref, m_ref, l_ref,
                  *, scale, bq, bk, hd, split_kv=False):
    qi = pl.program_id(1)
    ki = pl.program_id(2)

    @pl.when(ki == 0)
    def _():
        m_ref[...] = jnp.full((bq, 128), _NEG, jnp.float32)
        l_ref[...] = jnp.zeros((bq, 128), jnp.float32)
        acc_ref[...] = jnp.zeros((bq, hd), jnp.float32)

    def step(masked):
        if split_kv:
            kv = k_ref[...]
            kb = kv[:, :hd]
            vb = kv[:, hd:]
        else:
            kb = k_ref[...]
            vb = v_ref[...]
        s = lax.dot_general(
            q_ref[...].astype(jnp.bfloat16), kb.astype(jnp.bfloat16),
            (((1,), (1,)), ((), ())),
            preferred_element_type=jnp.float32) * scale
        if masked:
            row = lax.broadcasted_iota(jnp.int32, (bq, bk), 0)
            colg = lax.broadcasted_iota(jnp.int32, (bq, bk), 1)
            s = jnp.where(colg <= row, s, _NEG)

        m_prev = m_ref[...][:, :1]
        l_prev = l_ref[...][:, :1]
        m_cur = jnp.max(s, axis=1, keepdims=True)
        m_new = jnp.maximum(m_prev, m_cur)
        alpha = jnp.exp(m_prev - m_new)
        p = jnp.exp(s - m_new)
        l_new = l_prev * alpha + jnp.sum(p, axis=1, keepdims=True)
        acc_ref[...] = acc_ref[...] * alpha + lax.dot_general(
            p.astype(jnp.bfloat16), vb.astype(jnp.bfloat16),
            (((1,), (0,)), ((), ())),
            preferred_element_type=jnp.float32)
        m_ref[...] = jnp.broadcast_to(m_new, (bq, 128))
        l_ref[...] = jnp.broadcast_to(l_new, (bq, 128))
        if masked:
            o_ref[...] = acc_ref[...] / l_ref[...][:, :1]

    @pl.when(ki < qi)
    def _():
        step(False)

    @pl.when(ki == qi)
    def _():
        step(True)


def _flash2_kernel(q_ref, k_ref, v_ref, o_ref, acc_ref, m_ref, l_ref,
                   *, scale, bq, bk, hp):
    qi = pl.program_id(1)
    ki = pl.program_id(2)

    @pl.when(ki == 0)
    def _():
        m_ref[...] = jnp.full((bq, hp * _HD), _NEG, jnp.float32)
        l_ref[...] = jnp.zeros((bq, hp * _HD), jnp.float32)
        acc_ref[...] = jnp.zeros((bq, hp * _HD), jnp.float32)

    def step(masked):
        if masked:
            row = lax.broadcasted_iota(jnp.int32, (bq, bk), 0)
            colg = lax.broadcasted_iota(jnp.int32, (bq, bk), 1)
            keep = colg <= row
        qall = q_ref[...]
        kall = k_ref[...]
        vall = v_ref[...]
        for sub in range(hp):
            sl = slice(_HD * sub, _HD * (sub + 1))
            s = lax.dot_general(
                qall[:, sl], kall[:, sl],
                (((1,), (1,)), ((), ())),
                preferred_element_type=jnp.float32)
            if masked:
                s = jnp.where(keep, s, _NEG)
            m_prev = m_ref[:, _HD * sub:_HD * sub + 1]
            l_prev = l_ref[:, _HD * sub:_HD * sub + 1]
            m_cur = jnp.max(s, axis=1, keepdims=True)
            m_new = jnp.maximum(m_prev, m_cur)
            alpha = jnp.exp(m_prev - m_new)
            p = jnp.exp(s - m_new)
            l_new = l_prev * alpha + jnp.sum(p, axis=1, keepdims=True)
            acc_ref[:, sl] = acc_ref[:, sl] * alpha + lax.dot_general(
                p.astype(jnp.bfloat16), vall[:, sl],
                (((1,), (0,)), ((), ())),
                preferred_element_type=jnp.float32)
            m_ref[:, sl] = jnp.broadcast_to(m_new, (bq, _HD))
            l_ref[:, sl] = jnp.broadcast_to(l_new, (bq, _HD))
            if masked:
                o_ref[:, sl] = (acc_ref[:, sl]
                                / l_ref[:, _HD * sub:_HD * sub + 1]
                                ).astype(jnp.bfloat16)

    @pl.when(ki < qi)
    def _():
        step(False)

    @pl.when(ki == qi)
    def _():
        step(True)


def _base_attention(q, k, v):
    bq = bk = 512
    hp = 8
    nq = _S // bq
    kern = functools.partial(
        _flash2_kernel, scale=1.0 / math.sqrt(_HD), bq=bq, bk=bk, hp=hp)
    spec_q = pl.BlockSpec((bq, hp * _HD), lambda h, qi, ki: (qi, h))
    spec_kv = pl.BlockSpec((bk, hp * _HD),
                           lambda h, qi, ki: (jnp.minimum(ki, qi), h))
    return pl.pallas_call(
        kern,
        grid=(_NH // hp, nq, nq),
        in_specs=[spec_q, spec_kv, spec_kv],
        out_specs=pl.BlockSpec((bq, hp * _HD), lambda h, qi, ki: (qi, h)),
        out_shape=jax.ShapeDtypeStruct((_S, _H), jnp.bfloat16),
        scratch_shapes=[
            pltpu.VMEM((bq, hp * _HD), jnp.float32),
            pltpu.VMEM((bq, hp * _HD), jnp.float32),
            pltpu.VMEM((bq, hp * _HD), jnp.float32),
        ],
    )(q, k, v)


def _adapter_attention(qa, kvg):
    bq = bk = 512
    nq = _S // bq
    kern = functools.partial(
        _flash_kernel, scale=1.0 / math.sqrt(_D), bq=bq, bk=bk, hd=_D,
        split_kv=True)
    return pl.pallas_call(
        kern,
        grid=(1, nq, nq),
        in_specs=[
            pl.BlockSpec((bq, _D), lambda h, qi, ki: (qi, 0)),
            pl.BlockSpec((bk, 2 * _D),
                         lambda h, qi, ki: (jnp.minimum(ki, qi), 0)),
            pl.BlockSpec((bk, 2 * _D),
                         lambda h, qi, ki: (jnp.minimum(ki, qi), 0)),
        ],
        out_specs=pl.BlockSpec((bq, _D), lambda h, qi, ki: (qi, 0)),
        out_shape=jax.ShapeDtypeStruct((_S, _D), jnp.float32),
        scratch_shapes=[
            pltpu.VMEM((bq, _D), jnp.float32),
            pltpu.VMEM((bq, 128), jnp.float32),
            pltpu.VMEM((bq, 128), jnp.float32),
        ],
    )(qa, kvg, kvg)


def _final_kernel(attn_ref, wo_ref, mome_ref, wvi_ref, wvo_ref, o_ref):
    base = lax.dot_general(
        attn_ref[...], wo_ref[...].astype(jnp.bfloat16),
        (((1,), (1,)), ((), ())),
        preferred_element_type=jnp.float32)
    t = lax.dot_general(
        mome_ref[...], wvi_ref[...], (((1,), (1,)), ((), ())),
        preferred_element_type=jnp.float32)
    ad = lax.dot_general(
        t, wvo_ref[...], (((1,), (1,)), ((), ())),
        preferred_element_type=jnp.float32)
    o_ref[...] = base + ad * float(_SCALING)


def _final_proj(attn, wo, mome, wvi, wvo):
    cb = 512
    return pl.pallas_call(
        _final_kernel,
        grid=(_H // cb,),
        in_specs=[
            pl.BlockSpec((_S, _H), lambda i: (0, 0)),
            pl.BlockSpec((cb, _H), lambda i: (i, 0)),
            pl.BlockSpec((_S, _D), lambda i: (0, 0)),
            pl.BlockSpec((_R, _D), lambda i: (0, 0)),
            pl.BlockSpec((cb, _R), lambda i: (i, 0)),
        ],
        out_specs=pl.BlockSpec((_S, cb), lambda i: (0, i)),
        out_shape=jax.ShapeDtypeStruct((_S, _H), jnp.float32),
    )(attn, wo, mome, wvi, wvo)


def kernel(hidden_states, Wq, Wk, Wv, Wo, Wqi, Wqo, Wvi, Wvo,
           index_keys, index_values):
    x = hidden_states.reshape(_S, _H)

    q, k, v = _qkv_proj(x, Wq, Wk, Wv)
    qa = _qa_proj(x, Wqi, Wqo)

    catkv = jnp.concatenate([index_keys, index_values], axis=1)
    topidx = _retrieval_topk(qa[:_QROWS], catkv)
    flat_idx = topidx[:410, :_TOPK].reshape(-1)[:_S]

    kvg = _sc_gather(catkv, flat_idx)

    attn = _base_attention(q, k, v)
    mome = _adapter_attention(qa, kvg)

    out = _final_proj(attn, Wo, mome, Wvi, Wvo)
    return out.reshape(_B, _S, _H)


# speedup vs baseline: 4.2564x; 1.0042x over previous
import functools
import math

import jax
import jax.numpy as jnp
from jax import lax
from jax.experimental import pallas as pl
from jax.experimental.pallas import tpu as pltpu
from jax.experimental.pallas import tpu_sc as plsc

_B, _S, _H = 1, 2048, 2048
_NH = 16
_HD = _H // _NH
_R = 16
_D = 64
_K = 100000
_TOPK = 5
_SCALING = _R

_NEG = -1e30
_IBIG = 2**31 - 1

_QROWS = 416
_KBLK = 2000


def _qkv_kernel(x_ref, wq_ref, wk_ref, wv_ref, q_ref, k_ref, v_ref):
    xb = x_ref[...].astype(jnp.bfloat16)
    qscale = 1.0 / math.sqrt(_HD)
    for w_ref, o_ref, sc in ((wq_ref, q_ref, qscale), (wk_ref, k_ref, None),
                             (wv_ref, v_ref, None)):
        acc = lax.dot_general(
            xb, w_ref[...].astype(jnp.bfloat16), (((1,), (1,)), ((), ())),
            preferred_element_type=jnp.float32)
        if sc is not None:
            acc = acc * sc
        o_ref[...] = acc.astype(jnp.bfloat16)


def _qkv_proj(x, wq, wk, wv):
    cb = 256
    wspec = pl.BlockSpec((cb, _H), lambda i: (i, 0))
    ospec = pl.BlockSpec((_S, cb), lambda i: (0, i))
    oshape = jax.ShapeDtypeStruct((_S, _H), jnp.bfloat16)
    return pl.pallas_call(
        _qkv_kernel,
        grid=(_H // cb,),
        in_specs=[pl.BlockSpec((_S, _H), lambda i: (0, 0)),
                  wspec, wspec, wspec],
        out_specs=[ospec, ospec, ospec],
        out_shape=[oshape, oshape, oshape],
    )(x, wq, wk, wv)


def _qa_kernel(x_ref, wqi_ref, wqo_ref, o_ref):
    t = lax.dot_general(
        x_ref[...], wqi_ref[...], (((1,), (1,)), ((), ())),
        preferred_element_type=jnp.float32)
    o_ref[...] = lax.dot_general(
        t, wqo_ref[...], (((1,), (1,)), ((), ())),
        preferred_element_type=jnp.float32)


def _qa_proj(x, wqi, wqo):
    bs = 256
    return pl.pallas_call(
        _qa_kernel,
        grid=(_S // bs,),
        in_specs=[
            pl.BlockSpec((bs, _H), lambda i: (i, 0)),
            pl.BlockSpec((_R, _H), lambda i: (0, 0)),
            pl.BlockSpec((_D, _R), lambda i: (0, 0)),
        ],
        out_specs=pl.BlockSpec((bs, _D), lambda i: (i, 0)),
        out_shape=jax.ShapeDtypeStruct((_S, _D), jnp.float32),
    )(x, wqi, wqo)


def _topk_kernel(q_ref, keys_ref, oidx_ref, rv_ref, ri_ref, colf_ref):
    i = pl.program_id(0)
    nb = pl.num_programs(0)

    @pl.when(i == 0)
    def _():
        rv_ref[...] = jnp.full((_QROWS, 8), _NEG, jnp.float32)
        ri_ref[...] = jnp.full((_QROWS, 8), _IBIG, jnp.int32)
        colf_ref[...] = lax.broadcasted_iota(
            jnp.int32, (_QROWS, _KBLK), 1).astype(jnp.float32)

    s = lax.dot_general(
        q_ref[...], keys_ref[...][:, :_D], (((1,), (1,)), ((), ())),
        preferred_element_type=jnp.float32)
    colf = colf_ref[...]
    lane8 = lax.broadcasted_iota(jnp.int32, (_QROWS, 8), 1)

    bv = jnp.full((_QROWS, 8), _NEG, jnp.float32)
    bi = jnp.full((_QROWS, 8), _IBIG, jnp.int32)
    fbig = 3.0e8
    for t in range(_TOPK):
        v = jnp.max(s, axis=1, keepdims=True)
        idxf = jnp.min(jnp.where(s == v, colf, fbig), axis=1, keepdims=True)
        bv = jnp.where(lane8 == t, v, bv)
        bi = jnp.where(lane8 == t, idxf.astype(jnp.int32) + i * _KBLK, bi)
        s = jnp.where(colf == idxf, _NEG, s)

    cv = jnp.concatenate([rv_ref[...], bv], axis=1)
    ci = jnp.concatenate([ri_ref[...], bi], axis=1)
    nv = jnp.full((_QROWS, 8), _NEG, jnp.float32)
    ni = jnp.full((_QROWS, 8), _IBIG, jnp.int32)
    for t in range(_TOPK):
        v = jnp.max(cv, axis=1, keepdims=True)
        idx = jnp.min(jnp.where(cv == v, ci, _IBIG), axis=1, keepdims=True)
        nv = jnp.where(lane8 == t, v, nv)
        ni = jnp.where(lane8 == t, idx, ni)
        cv = jnp.where(ci == idx, _NEG, cv)
    rv_ref[...] = nv
    ri_ref[...] = ni

    @pl.when(i == nb - 1)
    def _():
        oidx_ref[...] = ri_ref[...]


def _retrieval_topk(qa_head, index_keys):
    return pl.pallas_call(
        _topk_kernel,
        grid=(_K // _KBLK,),
        in_specs=[
            pl.BlockSpec((_QROWS, _D), lambda i: (0, 0)),
            pl.BlockSpec((_KBLK, 2 * _D), lambda i: (i, 0)),
        ],
        out_specs=pl.BlockSpec((_QROWS, 8), lambda i: (0, 0)),
        out_shape=jax.ShapeDtypeStruct((_QROWS, 8), jnp.int32),
        scratch_shapes=[
            pltpu.VMEM((_QROWS, 8), jnp.float32),
            pltpu.VMEM((_QROWS, 8), jnp.int32),
            pltpu.VMEM((_QROWS, _KBLK), jnp.float32),
        ],
    )(qa_head, index_keys)


def _catkv_kernel(k_ref, v_ref, o_ref):
    o_ref[:, :_D] = k_ref[...]
    o_ref[:, _D:] = v_ref[...]


def _catkv_build(keys, values):
    blk = 4000
    return pl.pallas_call(
        _catkv_kernel,
        grid=(_K // blk,),
        in_specs=[
            pl.BlockSpec((blk, _D), lambda i: (i, 0)),
            pl.BlockSpec((blk, _D), lambda i: (i, 0)),
        ],
        out_specs=pl.BlockSpec((blk, 2 * _D), lambda i: (i, 0)),
        out_shape=jax.ShapeDtypeStruct((_K, 2 * _D), jnp.float32),
    )(keys, values)


def _sc_gather(catkv, idx):
    n = idx.shape[0]
    nw = 32
    per = n // nw
    mesh = plsc.VectorSubcoreMesh(core_axis_name="c", subcore_axis_name="s")

    @functools.partial(
        pl.kernel,
        mesh=mesh,
        out_type=jax.ShapeDtypeStruct((n, 2 * _D), jnp.float32),
        scratch_types=[
            pltpu.VMEM((per,), jnp.int32),
            pltpu.VMEM((per, 2 * _D), jnp.float32),
            pltpu.SemaphoreType.DMA,
        ],
    )
    def gk(cat_hbm, idx_hbm, o_hbm, idx_v, rows_v, sem):
        wid = lax.axis_index("s") * 2 + lax.axis_index("c")
        base = wid * per
        pltpu.sync_copy(idx_hbm.at[pl.ds(base, per)], idx_v)
        pltpu.async_copy(cat_hbm.at[idx_v], rows_v, sem).wait()
        pltpu.sync_copy(rows_v, o_hbm.at[pl.ds(base, per)])

    return gk(catkv, idx)


def _flash_kernel(q_ref, k_ref, v_ref, o_ref, acc_ref, m_ref, l_ref,
                  *, scale, bq, bk, hd, split_kv=False):
    qi = pl.program_id(1)
    ki = pl.program_id(2)

    @pl.when(ki == 0)
    def _():
        m_ref[...] = jnp.full((bq, 128), _NEG, jnp.float32)
        l_ref[...] = jnp.zeros((bq, 128), jnp.float32)
        acc_ref[...] = jnp.zeros((bq, hd), jnp.float32)

    def step(masked):
        if split_kv:
            kv = k_ref[...]
            kb = kv[:, :hd]
            vb = kv[:, hd:]
        else:
            kb = k_ref[...]
            vb = v_ref[...]
        s = lax.dot_general(
            q_ref[...].astype(jnp.bfloat16), kb.astype(jnp.bfloat16),
            (((1,), (1,)), ((), ())),
            preferred_element_type=jnp.float32) * scale
        if masked:
            row = lax.broadcasted_iota(jnp.int32, (bq, bk), 0)
            colg = lax.broadcasted_iota(jnp.int32, (bq, bk), 1)
            s = jnp.where(colg <= row, s, _NEG)

        m_prev = m_ref[...][:, :1]
        l_prev = l_ref[...][:, :1]
        m_cur = jnp.max(s, axis=1, keepdims=True)
        m_new = jnp.maximum(m_prev, m_cur)
        alpha = jnp.exp(m_prev - m_new)
        p = jnp.exp(s - m_new)
        l_new = l_prev * alpha + jnp.sum(p, axis=1, keepdims=True)
        acc_ref[...] = acc_ref[...] * alpha + lax.dot_general(
            p.astype(jnp.bfloat16), vb.astype(jnp.bfloat16),
            (((1,), (0,)), ((), ())),
            preferred_element_type=jnp.float32)
        m_ref[...] = jnp.broadcast_to(m_new, (bq, 128))
        l_ref[...] = jnp.broadcast_to(l_new, (bq, 128))
        if masked:
            o_ref[...] = acc_ref[...] / l_ref[...][:, :1]

    @pl.when(ki < qi)
    def _():
        step(False)

    @pl.when(ki == qi)
    def _():
        step(True)


def _flash2_kernel(q_ref, k_ref, v_ref, o_ref, acc_ref, m_ref, l_ref,
                   *, scale, bq, bk, hp):
    qi = pl.program_id(1)
    ki = pl.program_id(2)

    @pl.when(ki == 0)
    def _():
        m_ref[...] = jnp.full((bq, hp * _HD), _NEG, jnp.float32)
        l_ref[...] = jnp.zeros((bq, hp * _HD), jnp.float32)
        acc_ref[...] = jnp.zeros((bq, hp * _HD), jnp.float32)

    def step(masked):
        if masked:
            row = lax.broadcasted_iota(jnp.int32, (bq, bk), 0)
            colg = lax.broadcasted_iota(jnp.int32, (bq, bk), 1)
            keep = colg <= row
        qall = q_ref[...]
        kall = k_ref[...]
        vall = v_ref[...]
        for sub in range(hp):
            sl = slice(_HD * sub, _HD * (sub + 1))
            s = lax.dot_general(
                qall[:, sl], kall[:, sl],
                (((1,), (1,)), ((), ())),
                preferred_element_type=jnp.float32)
            if masked:
                s = jnp.where(keep, s, _NEG)
            m_prev = m_ref[:, _HD * sub:_HD * sub + 1]
            l_prev = l_ref[:, _HD * sub:_HD * sub + 1]
            m_cur = jnp.max(s, axis=1, keepdims=True)
            m_new = jnp.maximum(m_prev, m_cur)
            alpha = jnp.exp(m_prev - m_new)
            p = jnp.exp(s - m_new)
            l_new = l_prev * alpha + jnp.sum(p, axis=1, keepdims=True)
            acc_ref[:, sl] = acc_ref[:, sl] * alpha + lax.dot_general(
                p.astype(jnp.bfloat16), vall[:, sl],
                (((1,), (0,)), ((), ())),
                preferred_element_type=jnp.float32)
            m_ref[:, sl] = jnp.broadcast_to(m_new, (bq, _HD))
            l_ref[:, sl] = jnp.broadcast_to(l_new, (bq, _HD))
            if masked:
                o_ref[:, sl] = (acc_ref[:, sl]
                                / l_ref[:, _HD * sub:_HD * sub + 1]
                                ).astype(jnp.bfloat16)

    @pl.when(ki < qi)
    def _():
        step(False)

    @pl.when(ki == qi)
    def _():
        step(True)


def _base_attention(q, k, v):
    bq = bk = 512
    hp = 8
    nq = _S // bq
    kern = functools.partial(
        _flash2_kernel, scale=1.0 / math.sqrt(_HD), bq=bq, bk=bk, hp=hp)
    spec_q = pl.BlockSpec((bq, hp * _HD), lambda h, qi, ki: (qi, h))
    spec_kv = pl.BlockSpec((bk, hp * _HD),
                           lambda h, qi, ki: (jnp.minimum(ki, qi), h))
    return pl.pallas_call(
        kern,
        grid=(_NH // hp, nq, nq),
        in_specs=[spec_q, spec_kv, spec_kv],
        out_specs=pl.BlockSpec((bq, hp * _HD), lambda h, qi, ki: (qi, h)),
        out_shape=jax.ShapeDtypeStruct((_S, _H), jnp.bfloat16),
        scratch_shapes=[
            pltpu.VMEM((bq, hp * _HD), jnp.float32),
            pltpu.VMEM((bq, hp * _HD), jnp.float32),
            pltpu.VMEM((bq, hp * _HD), jnp.float32),
        ],
    )(q, k, v)


def _adapter_attention(qa, kvg):
    bq = bk = 512
    nq = _S // bq
    kern = functools.partial(
        _flash_kernel, scale=1.0 / math.sqrt(_D), bq=bq, bk=bk, hd=_D,
        split_kv=True)
    return pl.pallas_call(
        kern,
        grid=(1, nq, nq),
        in_specs=[
            pl.BlockSpec((bq, _D), lambda h, qi, ki: (qi, 0)),
            pl.BlockSpec((bk, 2 * _D),
                         lambda h, qi, ki: (jnp.minimum(ki, qi), 0)),
            pl.BlockSpec((bk, 2 * _D),
                         lambda h, qi, ki: (jnp.minimum(ki, qi), 0)),
        ],
        out_specs=pl.BlockSpec((bq, _D), lambda h, qi, ki: (qi, 0)),
        out_shape=jax.ShapeDtypeStruct((_S, _D), jnp.float32),
        scratch_shapes=[
            pltpu.VMEM((bq, _D), jnp.float32),
            pltpu.VMEM((bq, 128), jnp.float32),
            pltpu.VMEM((bq, 128), jnp.float32),
        ],
    )(qa, kvg, kvg)


def _final_kernel(attn_ref, wo_ref, mome_ref, wvi_ref, wvo_ref, o_ref):
    base = lax.dot_general(
        attn_ref[...], wo_ref[...].astype(jnp.bfloat16),
        (((1,), (1,)), ((), ())),
        preferred_element_type=jnp.float32)
    t = lax.dot_general(
        mome_ref[...], wvi_ref[...], (((1,), (1,)), ((), ())),
        preferred_element_type=jnp.float32)
    ad = lax.dot_general(
        t, wvo_ref[...], (((1,), (1,)), ((), ())),
        preferred_element_type=jnp.float32)
    o_ref[...] = base + ad * float(_SCALING)


def _final_proj(attn, wo, mome, wvi, wvo):
    cb = 512
    return pl.pallas_call(
        _final_kernel,
        grid=(_H // cb,),
        in_specs=[
            pl.BlockSpec((_S, _H), lambda i: (0, 0)),
            pl.BlockSpec((cb, _H), lambda i: (i, 0)),
            pl.BlockSpec((_S, _D), lambda i: (0, 0)),
            pl.BlockSpec((_R, _D), lambda i: (0, 0)),
            pl.BlockSpec((cb, _R), lambda i: (i, 0)),
        ],
        out_specs=pl.BlockSpec((_S, cb), lambda i: (0, i)),
        out_shape=jax.ShapeDtypeStruct((_S, _H), jnp.float32),
    )(attn, wo, mome, wvi, wvo)


def kernel(hidden_states, Wq, Wk, Wv, Wo, Wqi, Wqo, Wvi, Wvo,
           index_keys, index_values):
    x = hidden_states.reshape(_S, _H)

    q, k, v = _qkv_proj(x, Wq, Wk, Wv)
    qa = _qa_proj(x, Wqi, Wqo)

    catkv = jnp.concatenate([index_keys, index_values], axis=1)
    topidx = _retrieval_topk(qa[:_QROWS], catkv)
    flat_idx = topidx[:410, :_TOPK].reshape(-1)[:_S]

    kvg = _sc_gather(catkv, flat_idx)

    attn = _base_attention(q, k, v)
    mome = _adapter_attention(qa, kvg)

    out = _final_proj(attn, Wo, mome, Wvi, Wvo)
    return out.reshape(_B, _S, _H)
